# Initial kernel scaffold; baseline (speedup 1.0000x reference)
import jax
import jax.numpy as jnp
from jax.experimental import pallas as pl


def kernel(x, edge_index, log_dt, log_lambda_real, lambda_imag, W1, W2, Wp, bp):
    raise NotImplementedError("write your pallas kernel here")


# speedup vs baseline: 2.8981x; 2.8981x over previous
import jax
import jax.numpy as jnp
from jax import lax
from jax.experimental import pallas as pl
from jax.experimental.pallas import tpu as pltpu
from jax.experimental.pallas import tpu_sc as plsc

_N = 10000
_E = 160000
_D = 256
_HID = 512
_T = 3

_NC = 2
_NS = 16
_NW = _NC * _NS
_CH = _D // _NC

_GC = 40
_EPW = _E // _NW
_EPS = _E // _NS
_OWN = 640
_WB = 16

_mesh = plsc.VectorSubcoreMesh(core_axis_name="c", subcore_axis_name="s")


def _fill(ref, nrows, ncols, val):
    v = jnp.full((16,), val, jnp.float32)
    per = ncols // 16

    def st(t, carry):
        ref[t // per, pl.ds((t % per) * 16, 16)] = v
        return carry

    lax.fori_loop(0, nrows * per, st, 0)


def _gather_body(tab, idx, out, idxv, rowsv, sem):
    c = lax.axis_index("c")
    s = lax.axis_index("s")
    base = (s * _NC + c) * _EPW

    def step(k, carry):
        e0 = pl.multiple_of(base + k * _GC, 8)
        pltpu.sync_copy(idx.at[pl.ds(e0, _GC)], idxv)
        pltpu.async_copy(tab.at[idxv], rowsv, sem).wait()
        pltpu.sync_copy(rowsv, out.at[pl.ds(e0, _GC)])
        return carry

    lax.fori_loop(0, _EPW // _GC, step, 0)


_gather = pl.kernel(
    _gather_body,
    out_type=jax.ShapeDtypeStruct((_E, _D), jnp.float32),
    mesh=_mesh,
    scratch_types=[
        pltpu.VMEM((_GC,), jnp.int32),
        pltpu.VMEM((_GC, _D), jnp.float32),
        pltpu.SemaphoreType.DMA,
    ],
)


def _scatter_body(mrows, dstidx, out, idxv, rowsv, stage, macc):
    c = lax.axis_index("c")
    s = lax.axis_index("s")

    row0 = s * _OWN
    nch = jnp.where(s == _NS - 1, (_N - (_NS - 1) * _OWN) // _WB, _OWN // _WB)

    _fill(stage, _WB, _CH, 0.0)

    def zstep(k, carry):
        pltpu.sync_copy(stage, macc.at[pl.ds(pl.multiple_of(row0 + k * _WB, 8), _WB)])
        return carry

    lax.fori_loop(0, nch, zstep, 0)
    plsc.subcore_barrier()

    base = s * _EPS

    def step(k, carry):
        e0 = pl.multiple_of(base + k * _GC, 8)
        pltpu.sync_copy(dstidx.at[pl.ds(e0, _GC)], idxv)
        pltpu.sync_copy(mrows.at[pl.ds(e0, _GC), pl.ds(c * _CH, _CH)], rowsv)
        pltpu.sync_copy(rowsv, macc.at[idxv], add=True)
        return carry

    lax.fori_loop(0, _EPS // _GC, step, 0)
    plsc.subcore_barrier()

    def wstep(k, carry):
        r0 = pl.multiple_of(row0 + k * _WB, 8)
        pltpu.sync_copy(macc.at[pl.ds(r0, _WB)], stage)
        pltpu.sync_copy(stage, out.at[pl.ds(r0, _WB), pl.ds(c * _CH, _CH)])
        return carry

    lax.fori_loop(0, nch, wstep, 0)


_scatter = pl.kernel(
    _scatter_body,
    out_type=jax.ShapeDtypeStruct((_N, _D), jnp.float32),
    mesh=_mesh,
    scratch_types=[
        pltpu.VMEM((_GC,), jnp.int32),
        pltpu.VMEM((_GC, _CH), jnp.float32),
        pltpu.VMEM((_WB, _CH), jnp.float32),
        pltpu.VMEM_SHARED((_N, _CH), jnp.float32),
    ],
)


def _deg_body(srcidx, out, idxv, rowsv, stage, macc):
    c = lax.axis_index("c")
    s = lax.axis_index("s")

    row0 = s * _OWN
    nch = jnp.where(s == _NS - 1, (_N - (_NS - 1) * _OWN) // _WB, _OWN // _WB)

    _fill(stage, _WB, _CH, 0.0)

    def zstep(k, carry):
        pltpu.sync_copy(stage, macc.at[pl.ds(pl.multiple_of(row0 + k * _WB, 8), _WB)])
        return carry

    lax.fori_loop(0, nch, zstep, 0)
    _fill(rowsv, _GC, _CH, 1.0)
    plsc.subcore_barrier()

    base = s * _EPS

    def step(k, carry):
        e0 = pl.multiple_of(base + k * _GC, 8)
        pltpu.sync_copy(srcidx.at[pl.ds(e0, _GC)], idxv)
        pltpu.sync_copy(rowsv, macc.at[idxv], add=True)
        return carry

    lax.fori_loop(0, _EPS // _GC, step, 0)
    plsc.subcore_barrier()

    def wstep(k, carry):
        r0 = pl.multiple_of(row0 + k * _WB, 8)
        pltpu.sync_copy(macc.at[pl.ds(r0, _WB)], stage)
        pltpu.sync_copy(stage, out.at[pl.ds(r0, _WB), pl.ds(c * _CH, _CH)])
        return carry

    lax.fori_loop(0, nch, wstep, 0)


_deg = pl.kernel(
    _deg_body,
    out_type=jax.ShapeDtypeStruct((_N, _D), jnp.float32),
    mesh=_mesh,
    scratch_types=[
        pltpu.VMEM((_GC,), jnp.int32),
        pltpu.VMEM((_GC, _CH), jnp.float32),
        pltpu.VMEM((_WB, _CH), jnp.float32),
        pltpu.VMEM_SHARED((_N, _CH), jnp.float32),
    ],
)


_BLK_N = 400
_CN = (((1,), (1,)), ((), ()))


def _mlp_body(xb, w1, w2, wp, bpb, ob):
    xx = xb[...]
    a = lax.dot_general(xx, w1[...], _CN, preferred_element_type=jnp.float32)
    g = lax.dot_general(xx, w2[...], _CN, preferred_element_type=jnp.float32)
    hh = (a * jax.nn.sigmoid(a)) * g
    ob[...] = lax.dot_general(hh, wp[...], _CN, preferred_element_type=jnp.float32) + bpb[...]


def _mlp(x, W1, W2, Wp, bp):
    return pl.pallas_call(
        _mlp_body,
        grid=(_N // _BLK_N,),
        in_specs=[
            pl.BlockSpec((_BLK_N, _D), lambda i: (i, 0)),
            pl.BlockSpec((_HID, _D), lambda i: (0, 0)),
            pl.BlockSpec((_HID, _D), lambda i: (0, 0)),
            pl.BlockSpec((_D, _HID), lambda i: (0, 0)),
            pl.BlockSpec((1, _D), lambda i: (0, 0)),
        ],
        out_specs=pl.BlockSpec((_BLK_N, _D), lambda i: (i, 0)),
        out_shape=jax.ShapeDtypeStruct((_N, _D), jnp.float32),
    )(x, W1, W2, Wp, bp.reshape(1, _D))


def _swap(v):
    ev = (lax.broadcasted_iota(jnp.int32, v.shape, 1) % 2) == 0
    return jnp.where(ev, pltpu.roll(v, _D - 1, 1), pltpu.roll(v, 1, 1))


_BLK_E = 400
_NB_E = _E // _BLK_E
_HALF = (_E // 2) // _BLK_E


def _combine_body(gh, gm, mprev, gdeg, ca, sa, cb, sb, out):
    xe = gh[...]
    de = gdeg[...]
    dm = gm[...] - mprev[...]
    la = ca[...] * xe + sa[...] * _swap(xe)
    lb = cb[...] * dm + sb[...] * _swap(dm)
    out[...] = jnp.where(de == 1.0, xe, la + lb / (de - 1.0 + 1e-9))


def _combine(gh, gm, mprev, gdeg, CA, SA, CB, SB):
    coef = pl.BlockSpec((1, _D), lambda i: (0, 0))
    eb = pl.BlockSpec((_BLK_E, _D), lambda i: (i, 0))
    return pl.pallas_call(
        _combine_body,
        grid=(_NB_E,),
        in_specs=[
            eb, eb,
            pl.BlockSpec((_BLK_E, _D), lambda i: ((i + _HALF) % _NB_E, 0)),
            eb, coef, coef, coef, coef,
        ],
        out_specs=eb,
        out_shape=jax.ShapeDtypeStruct((_E, _D), jnp.float32),
    )(gh, gm, mprev, gdeg, CA, SA, CB, SB)


def _final_body(xb, hb, mb, degb, ca, sa, cb, sb, out):
    hh = hb[...]
    mm = mb[...]
    dg = degb[...]
    la = ca[...] * hh + sa[...] * _swap(hh)
    lb = cb[...] * mm + sb[...] * _swap(mm)
    val = jnp.where(dg == 0.0, hh, la + lb / (dg + 1e-9))
    out[...] = xb[...] + jnp.maximum(val, 0.0)


def _final(x, h, m, degf, CA, SA, CB, SB):
    coef = pl.BlockSpec((1, _D), lambda i: (0, 0))
    nb = pl.BlockSpec((_BLK_N, _D), lambda i: (i, 0))
    return pl.pallas_call(
        _final_body,
        grid=(_N // _BLK_N,),
        in_specs=[nb, nb, nb, nb, coef, coef, coef, coef],
        out_specs=nb,
        out_shape=jax.ShapeDtypeStruct((_N, _D), jnp.float32),
    )(x, h, m, degf, CA, SA, CB, SB)


def kernel(x, edge_index, log_dt, log_lambda_real, lambda_imag, W1, W2, Wp, bp):
    src = edge_index[0]
    dst = edge_index[1]

    dt = jnp.exp(log_dt)
    mag = jnp.exp(-jnp.exp(log_lambda_real) * dt)
    ph = lambda_imag * dt
    cw = jnp.cos(ph)
    sw = jnp.sin(ph)

    def coeffs(mx):
        cc = jnp.repeat(mx * cw, 2).reshape(1, _D)
        ss = jnp.stack((-mx * sw, mx * sw), axis=-1).reshape(1, _D)
        return cc, ss

    CA, SA = coeffs(1.0 - mag)
    CB, SB = coeffs(mag)

    h = _mlp(x, W1, W2, Wp, bp)
    degf = _deg(src)
    gh = _gather(h, src)
    gdeg = _gather(degf, src)

    M = gh
    m = _scatter(gh, dst)
    for _ in range(_T):
        gm = _gather(m, src)
        M = _combine(gh, gm, M, gdeg, CA, SA, CB, SB)
        m = _scatter(M, dst)

    return _final(x, h, m, degf, CA, SA, CB, SB)


# speedup vs baseline: 4.6426x; 1.6019x over previous
import jax
import jax.numpy as jnp
from jax import lax
from jax.experimental import pallas as pl
from jax.experimental.pallas import tpu as pltpu
from jax.experimental.pallas import tpu_sc as plsc

_N = 10000
_E = 160000
_D = 256
_HID = 512
_T = 3

_NC = 2
_NS = 16
_NW = _NC * _NS
_CH = _D // _NC
_DG = 128

_EPW = _E // _NW
_EPS = _E // _NS
_GC = 40
_SCC = 80
_OWN = 640
_WB = 80

_mesh = plsc.VectorSubcoreMesh(core_axis_name="c", subcore_axis_name="s")


def _fill(ref, nrows, ncols, val):
    v = jnp.full((16,), val, jnp.float32)
    per = ncols // 16

    def st(t, carry):
        ref[t // per, pl.ds((t % per) * 16, 16)] = v
        return carry

    lax.fori_loop(0, nrows * per, st, 0)


def _own_rows(s):
    row0 = s * _OWN
    nch = jnp.where(s == _NS - 1, (_N - (_NS - 1) * _OWN) // _WB, _OWN // _WB)
    return row0, nch


def _gather_body(tab, idx, out, ic0, ic1, r0, r1, s0, s1, w0, w1, si0, si1):
    c = lax.axis_index("c")
    s = lax.axis_index("s")
    base = (s * _NC + c) * _EPW

    def pair(kk, carry):
        k0 = 2 * kk
        e0 = pl.multiple_of(base + k0 * _GC, 8)
        e1 = pl.multiple_of(e0 + _GC, 8)
        i0 = pltpu.async_copy(idx.at[pl.ds(e0, _GC)], ic0, si0)
        i1 = pltpu.async_copy(idx.at[pl.ds(e1, _GC)], ic1, si1)
        i0.wait()
        g0 = pltpu.async_copy(tab.at[ic0], r0, s0)
        i1.wait()
        g1 = pltpu.async_copy(tab.at[ic1], r1, s1)
        g0.wait()
        wa = pltpu.async_copy(r0, out.at[pl.ds(e0, _GC)], w0)
        g1.wait()
        wb = pltpu.async_copy(r1, out.at[pl.ds(e1, _GC)], w1)
        wa.wait()
        wb.wait()
        return carry

    lax.fori_loop(0, (_EPW // _GC) // 2, pair, 0)
    et = pl.multiple_of(base + (_EPW // _GC - 1) * _GC, 8)
    pltpu.sync_copy(idx.at[pl.ds(et, _GC)], ic0)
    pltpu.async_copy(tab.at[ic0], r0, s0).wait()
    pltpu.sync_copy(r0, out.at[pl.ds(et, _GC)])


_gather = pl.kernel(
    _gather_body,
    out_type=jax.ShapeDtypeStruct((_E, _D), jnp.float32),
    mesh=_mesh,
    scratch_types=[
        pltpu.VMEM((_GC,), jnp.int32),
        pltpu.VMEM((_GC,), jnp.int32),
        pltpu.VMEM((_GC, _D), jnp.float32),
        pltpu.VMEM((_GC, _D), jnp.float32),
        pltpu.SemaphoreType.DMA,
        pltpu.SemaphoreType.DMA,
        pltpu.SemaphoreType.DMA,
        pltpu.SemaphoreType.DMA,
        pltpu.SemaphoreType.DMA,
        pltpu.SemaphoreType.DMA,
    ],
)


def _seg_phase1(mrows, dstidx, c, s, ic0, ic1, r0, r1, s0, s1, a0, a1, si0, si1, macc):
    row0, nch = _own_rows(s)
    _fill(r0, _WB, _CH, 0.0)

    def zstep(k, carry):
        pltpu.sync_copy(r0, macc.at[pl.ds(pl.multiple_of(row0 + k * _WB, 8), _WB)])
        return carry

    lax.fori_loop(0, nch, zstep, 0)
    plsc.subcore_barrier()

    base = s * _EPS
    ccol = c * _CH

    def pair(kk, carry):
        k0 = 2 * kk
        e0 = pl.multiple_of(base + k0 * _SCC, 8)
        e1 = pl.multiple_of(e0 + _SCC, 8)
        i0 = pltpu.async_copy(dstidx.at[pl.ds(e0, _SCC)], ic0, si0)
        i1 = pltpu.async_copy(dstidx.at[pl.ds(e1, _SCC)], ic1, si1)
        l0 = pltpu.async_copy(mrows.at[pl.ds(e0, _SCC), pl.ds(ccol, _CH)], r0, s0)
        l1 = pltpu.async_copy(mrows.at[pl.ds(e1, _SCC), pl.ds(ccol, _CH)], r1, s1)
        i0.wait()
        l0.wait()
        x0 = pltpu.async_copy(r0, macc.at[ic0], a0, add=True)
        i1.wait()
        l1.wait()
        x1 = pltpu.async_copy(r1, macc.at[ic1], a1, add=True)
        x0.wait()
        x1.wait()
        return carry

    lax.fori_loop(0, (_EPS // _SCC) // 2, pair, 0)
    et = pl.multiple_of(base + (_EPS // _SCC - 1) * _SCC, 8)
    pltpu.sync_copy(mrows.at[pl.ds(et, _SCC), pl.ds(ccol, _CH)], r0)
    pltpu.sync_copy(dstidx.at[pl.ds(et, _SCC)], ic0)
    pltpu.sync_copy(r0, macc.at[ic0], add=True)
    plsc.subcore_barrier()


def _seg_writeback(out, c, s, r0, macc):
    row0, nch = _own_rows(s)

    def wstep(k, carry):
        rr = pl.multiple_of(row0 + k * _WB, 8)
        pltpu.sync_copy(macc.at[pl.ds(rr, _WB)], r0)
        pltpu.sync_copy(r0, out.at[pl.ds(rr, _WB), pl.ds(c * _CH, _CH)])
        return carry

    lax.fori_loop(0, nch, wstep, 0)


def _segsum_body(mrows, dstidx, out, ic0, ic1, r0, r1, s0, s1, a0, a1, si0, si1, macc):
    c = lax.axis_index("c")
    s = lax.axis_index("s")
    _seg_phase1(mrows, dstidx, c, s, ic0, ic1, r0, r1, s0, s1, a0, a1, si0, si1, macc)
    _seg_writeback(out, c, s, r0, macc)


_seg_scratch = [
    pltpu.VMEM((_SCC,), jnp.int32),
    pltpu.VMEM((_SCC,), jnp.int32),
    pltpu.VMEM((_SCC, _CH), jnp.float32),
    pltpu.VMEM((_SCC, _CH), jnp.float32),
    pltpu.SemaphoreType.DMA,
    pltpu.SemaphoreType.DMA,
    pltpu.SemaphoreType.DMA,
    pltpu.SemaphoreType.DMA,
    pltpu.SemaphoreType.DMA,
    pltpu.SemaphoreType.DMA,
    pltpu.VMEM_SHARED((_N, _CH), jnp.float32),
]

_segsum = pl.kernel(
    _segsum_body,
    out_type=jax.ShapeDtypeStruct((_N, _D), jnp.float32),
    mesh=_mesh,
    scratch_types=_seg_scratch,
)


def _segsum_gather_body(mrows, dstidx, srcidx, out, gmout,
                        ic0, ic1, r0, r1, s0, s1, a0, a1, si0, si1, macc):
    c = lax.axis_index("c")
    s = lax.axis_index("s")
    _seg_phase1(mrows, dstidx, c, s, ic0, ic1, r0, r1, s0, s1, a0, a1, si0, si1, macc)

    base = s * _EPS
    ccol = c * _CH

    def pair(kk, carry):
        k0 = 2 * kk
        e0 = pl.multiple_of(base + k0 * _SCC, 8)
        e1 = pl.multiple_of(e0 + _SCC, 8)
        i0 = pltpu.async_copy(srcidx.at[pl.ds(e0, _SCC)], ic0, si0)
        i1 = pltpu.async_copy(srcidx.at[pl.ds(e1, _SCC)], ic1, si1)
        i0.wait()
        g0 = pltpu.async_copy(macc.at[ic0], r0, s0)
        i1.wait()
        g1 = pltpu.async_copy(macc.at[ic1], r1, s1)
        g0.wait()
        wa = pltpu.async_copy(r0, gmout.at[pl.ds(e0, _SCC), pl.ds(ccol, _CH)], a0)
        g1.wait()
        wb = pltpu.async_copy(r1, gmout.at[pl.ds(e1, _SCC), pl.ds(ccol, _CH)], a1)
        wa.wait()
        wb.wait()
        return carry

    lax.fori_loop(0, (_EPS // _SCC) // 2, pair, 0)
    et = pl.multiple_of(base + (_EPS // _SCC - 1) * _SCC, 8)
    pltpu.sync_copy(srcidx.at[pl.ds(et, _SCC)], ic0)
    pltpu.async_copy(macc.at[ic0], r0, s0).wait()
    pltpu.sync_copy(r0, gmout.at[pl.ds(et, _SCC), pl.ds(ccol, _CH)])

    _seg_writeback(out, c, s, r0, macc)


_segsum_gather = pl.kernel(
    _segsum_gather_body,
    out_type=(
        jax.ShapeDtypeStruct((_N, _D), jnp.float32),
        jax.ShapeDtypeStruct((_E, _D), jnp.float32),
    ),
    mesh=_mesh,
    scratch_types=_seg_scratch,
)


def _deg_body(srcidx, out, gdout, ic0, ic1, icg0, icg1, ones, gr0, gr1, st, s0, s1, a0, a1, si0, si1, dacc):
    c = lax.axis_index("c")
    s = lax.axis_index("s")
    row0, nch = _own_rows(s)
    _fill(st, _WB, _DG, 0.0)
    _fill(ones, _SCC, _DG, 1.0)

    def zstep(k, carry):
        pltpu.sync_copy(st, dacc.at[pl.ds(pl.multiple_of(row0 + k * _WB, 8), _WB)])
        return carry

    lax.fori_loop(0, nch, zstep, 0)
    plsc.subcore_barrier()

    base = s * _EPS

    def pair(kk, carry):
        k0 = 2 * kk
        e0 = pl.multiple_of(base + k0 * _SCC, 8)
        e1 = pl.multiple_of(e0 + _SCC, 8)
        i0 = pltpu.async_copy(srcidx.at[pl.ds(e0, _SCC)], ic0, si0)
        i1 = pltpu.async_copy(srcidx.at[pl.ds(e1, _SCC)], ic1, si1)
        i0.wait()
        x0 = pltpu.async_copy(ones, dacc.at[ic0], a0, add=True)
        i1.wait()
        x1 = pltpu.async_copy(ones, dacc.at[ic1], a1, add=True)
        x0.wait()
        x1.wait()
        return carry

    lax.fori_loop(0, (_EPS // _SCC) // 2, pair, 0)
    et = pl.multiple_of(base + (_EPS // _SCC - 1) * _SCC, 8)
    pltpu.sync_copy(srcidx.at[pl.ds(et, _SCC)], ic0)
    pltpu.sync_copy(ones, dacc.at[ic0], add=True)
    plsc.subcore_barrier()

    gbase = (s * _NC + c) * _EPW

    def gpair(kk, carry):
        k0 = 2 * kk
        e0 = pl.multiple_of(gbase + k0 * _GC, 8)
        e1 = pl.multiple_of(e0 + _GC, 8)
        i0 = pltpu.async_copy(srcidx.at[pl.ds(e0, _GC)], icg0, si0)
        i1 = pltpu.async_copy(srcidx.at[pl.ds(e1, _GC)], icg1, si1)
        i0.wait()
        g0 = pltpu.async_copy(dacc.at[icg0], gr0, s0)
        i1.wait()
        g1 = pltpu.async_copy(dacc.at[icg1], gr1, s1)
        g0.wait()
        wa = pltpu.async_copy(gr0, gdout.at[pl.ds(e0, _GC)], a0)
        g1.wait()
        wb = pltpu.async_copy(gr1, gdout.at[pl.ds(e1, _GC)], a1)
        wa.wait()
        wb.wait()
        return carry

    lax.fori_loop(0, (_EPW // _GC) // 2, gpair, 0)
    et = pl.multiple_of(gbase + (_EPW // _GC - 1) * _GC, 8)
    pltpu.sync_copy(srcidx.at[pl.ds(et, _GC)], icg0)
    pltpu.async_copy(dacc.at[icg0], gr0, s0).wait()
    pltpu.sync_copy(gr0, gdout.at[pl.ds(et, _GC)])

    @pl.when(c == 0)
    def _():
        def wstep(k, carry):
            rr = pl.multiple_of(row0 + k * _WB, 8)
            pltpu.sync_copy(dacc.at[pl.ds(rr, _WB)], st)
            pltpu.sync_copy(st, out.at[pl.ds(rr, _WB)])
            return carry

        lax.fori_loop(0, nch, wstep, 0)


_deg = pl.kernel(
    _deg_body,
    out_type=(
        jax.ShapeDtypeStruct((_N, _DG), jnp.float32),
        jax.ShapeDtypeStruct((_E, _DG), jnp.float32),
    ),
    mesh=_mesh,
    scratch_types=[
        pltpu.VMEM((_SCC,), jnp.int32),
        pltpu.VMEM((_SCC,), jnp.int32),
        pltpu.VMEM((_GC,), jnp.int32),
        pltpu.VMEM((_GC,), jnp.int32),
        pltpu.VMEM((_SCC, _DG), jnp.float32),
        pltpu.VMEM((_GC, _DG), jnp.float32),
        pltpu.VMEM((_GC, _DG), jnp.float32),
        pltpu.VMEM((_WB, _DG), jnp.float32),
        pltpu.SemaphoreType.DMA,
        pltpu.SemaphoreType.DMA,
        pltpu.SemaphoreType.DMA,
        pltpu.SemaphoreType.DMA,
        pltpu.SemaphoreType.DMA,
        pltpu.SemaphoreType.DMA,
        pltpu.VMEM_SHARED((_N, _DG), jnp.float32),
    ],
)


_BLK_N = 400
_CN = (((1,), (1,)), ((), ()))


def _mlp_body(xb, w1, w2, wp, bpb, ob):
    xx = xb[...]
    a = lax.dot_general(xx, w1[...], _CN, preferred_element_type=jnp.float32)
    g = lax.dot_general(xx, w2[...], _CN, preferred_element_type=jnp.float32)
    hh = (a * jax.nn.sigmoid(a)) * g
    ob[...] = lax.dot_general(hh, wp[...], _CN, preferred_element_type=jnp.float32) + bpb[...]


def _mlp(x, W1, W2, Wp, bp):
    return pl.pallas_call(
        _mlp_body,
        grid=(_N // _BLK_N,),
        in_specs=[
            pl.BlockSpec((_BLK_N, _D), lambda i: (i, 0)),
            pl.BlockSpec((_HID, _D), lambda i: (0, 0)),
            pl.BlockSpec((_HID, _D), lambda i: (0, 0)),
            pl.BlockSpec((_D, _HID), lambda i: (0, 0)),
            pl.BlockSpec((1, _D), lambda i: (0, 0)),
        ],
        out_specs=pl.BlockSpec((_BLK_N, _D), lambda i: (i, 0)),
        out_shape=jax.ShapeDtypeStruct((_N, _D), jnp.float32),
    )(x, W1, W2, Wp, bp.reshape(1, _D))


def _swap(v):
    ev = (lax.broadcasted_iota(jnp.int32, v.shape, 1) % 2) == 0
    return jnp.where(ev, pltpu.roll(v, _D - 1, 1), pltpu.roll(v, 1, 1))


_BLK_E = 400
_NB_E = _E // _BLK_E
_HALF = (_E // 2) // _BLK_E


def _combine_body(gh, gm, mprev, gdeg, ca, sa, cb, sb, out):
    xe = gh[...]
    de = gdeg[...][:, :1]
    dm = gm[...] - mprev[...]
    la = ca[...] * xe + sa[...] * _swap(xe)
    lb = cb[...] * dm + sb[...] * _swap(dm)
    out[...] = jnp.where(de == 1.0, xe, la + lb / (de - 1.0 + 1e-9))


def _combine(gh, gm, mprev, gdeg, CA, SA, CB, SB):
    coef = pl.BlockSpec((1, _D), lambda i: (0, 0))
    eb = pl.BlockSpec((_BLK_E, _D), lambda i: (i, 0))
    return pl.pallas_call(
        _combine_body,
        grid=(_NB_E,),
        in_specs=[
            eb, eb,
            pl.BlockSpec((_BLK_E, _D), lambda i: ((i + _HALF) % _NB_E, 0)),
            pl.BlockSpec((_BLK_E, _DG), lambda i: (i, 0)),
            coef, coef, coef, coef,
        ],
        out_specs=eb,
        out_shape=jax.ShapeDtypeStruct((_E, _D), jnp.float32),
    )(gh, gm, mprev, gdeg, CA, SA, CB, SB)


def _final_body(xb, hb, mb, degb, ca, sa, cb, sb, out):
    hh = hb[...]
    mm = mb[...]
    dg = degb[...][:, :1]
    la = ca[...] * hh + sa[...] * _swap(hh)
    lb = cb[...] * mm + sb[...] * _swap(mm)
    val = jnp.where(dg == 0.0, hh, la + lb / (dg + 1e-9))
    out[...] = xb[...] + jnp.maximum(val, 0.0)


def _final(x, h, m, degn, CA, SA, CB, SB):
    coef = pl.BlockSpec((1, _D), lambda i: (0, 0))
    nb = pl.BlockSpec((_BLK_N, _D), lambda i: (i, 0))
    return pl.pallas_call(
        _final_body,
        grid=(_N // _BLK_N,),
        in_specs=[
            nb, nb, nb,
            pl.BlockSpec((_BLK_N, _DG), lambda i: (i, 0)),
            coef, coef, coef, coef,
        ],
        out_specs=nb,
        out_shape=jax.ShapeDtypeStruct((_N, _D), jnp.float32),
    )(x, h, m, degn, CA, SA, CB, SB)


def kernel(x, edge_index, log_dt, log_lambda_real, lambda_imag, W1, W2, Wp, bp):
    src = edge_index[0]
    dst = edge_index[1]

    dt = jnp.exp(log_dt)
    mag = jnp.exp(-jnp.exp(log_lambda_real) * dt)
    ph = lambda_imag * dt
    cw = jnp.cos(ph)
    sw = jnp.sin(ph)

    def coeffs(mx):
        cc = jnp.repeat(mx * cw, 2).reshape(1, _D)
        ss = jnp.stack((-mx * sw, mx * sw), axis=-1).reshape(1, _D)
        return cc, ss

    CA, SA = coeffs(1.0 - mag)
    CB, SB = coeffs(mag)

    h = _mlp(x, W1, W2, Wp, bp)
    degn, gdeg = _deg(src)
    gh = _gather(h, src)

    M = gh
    m, gm = _segsum_gather(gh, dst, src)
    for t in range(_T):
        M = _combine(gh, gm, M, gdeg, CA, SA, CB, SB)
        if t < _T - 1:
            m, gm = _segsum_gather(M, dst, src)
        else:
            m = _segsum(M, dst)

    return _final(x, h, m, degn, CA, SA, CB, SB)


# speedup vs baseline: 4.9119x; 1.0580x over previous
import jax
import jax.numpy as jnp
from jax import lax
from jax.experimental import pallas as pl
from jax.experimental.pallas import tpu as pltpu
from jax.experimental.pallas import tpu_sc as plsc

_N = 10000
_E = 160000
_D = 256
_HID = 512
_T = 3

_NC = 2
_NS = 16
_NW = _NC * _NS
_CH = _D // _NC
_DG = 128

_EPW = _E // _NW
_EPS = _E // _NS
_GC = 40
_SCC = 80
_C = 128
_NCHK = _E // _C
_OWN = 640
_WB = 80

_mesh = plsc.VectorSubcoreMesh(core_axis_name="c", subcore_axis_name="s")


def _fill(ref, nrows, ncols, val):
    v = jnp.full((16,), val, jnp.float32)
    per = ncols // 16

    def st(t, carry):
        ref[t // per, pl.ds((t % per) * 16, 16)] = v
        return carry

    lax.fori_loop(0, nrows * per, st, 0)


def _own_rows(s):
    row0 = s * _OWN
    nch = jnp.where(s == _NS - 1, (_N - (_NS - 1) * _OWN) // _WB, _OWN // _WB)
    return row0, nch


def _gather_body(tab, idx, out, ic0, ic1, r0, r1, s0, s1, w0, w1, si0, si1):
    c = lax.axis_index("c")
    s = lax.axis_index("s")
    wid = s * _NC + c
    nk = _NCHK // _NW + jnp.where(wid < _NCHK % _NW, 1, 0)

    def chunk1(j):
        e0 = pl.multiple_of((wid + j * _NW) * _C, 8)
        pltpu.sync_copy(idx.at[pl.ds(e0, _C)], ic0)
        pltpu.async_copy(tab.at[ic0], r0, s0).wait()
        pltpu.sync_copy(r0, out.at[pl.ds(e0, _C)])

    def pair(jj, carry):
        j0 = 2 * jj
        e0 = pl.multiple_of((wid + j0 * _NW) * _C, 8)
        e1 = pl.multiple_of((wid + (j0 + 1) * _NW) * _C, 8)
        i0 = pltpu.async_copy(idx.at[pl.ds(e0, _C)], ic0, si0)
        i1 = pltpu.async_copy(idx.at[pl.ds(e1, _C)], ic1, si1)
        i0.wait()
        g0 = pltpu.async_copy(tab.at[ic0], r0, s0)
        i1.wait()
        g1 = pltpu.async_copy(tab.at[ic1], r1, s1)
        g0.wait()
        wa = pltpu.async_copy(r0, out.at[pl.ds(e0, _C)], w0)
        g1.wait()
        wb = pltpu.async_copy(r1, out.at[pl.ds(e1, _C)], w1)
        wa.wait()
        wb.wait()
        return carry

    lax.fori_loop(0, nk // 2, pair, 0)

    @pl.when(nk % 2 == 1)
    def _():
        chunk1(nk - 1)


_gather = pl.kernel(
    _gather_body,
    out_type=jax.ShapeDtypeStruct((_E, _D), jnp.float32),
    mesh=_mesh,
    scratch_types=[
        pltpu.VMEM((_C,), jnp.int32),
        pltpu.VMEM((_C,), jnp.int32),
        pltpu.VMEM((_C, _D), jnp.float32),
        pltpu.VMEM((_C, _D), jnp.float32),
        pltpu.SemaphoreType.DMA,
        pltpu.SemaphoreType.DMA,
        pltpu.SemaphoreType.DMA,
        pltpu.SemaphoreType.DMA,
        pltpu.SemaphoreType.DMA,
        pltpu.SemaphoreType.DMA,
    ],
)


def _seg_phase1(mrows, dstidx, c, s, ic0, ic1, r0, r1, s0, s1, a0, a1, si0, si1, macc):
    row0, nch = _own_rows(s)
    _fill(r0, _WB, _CH, 0.0)

    def zstep(k, carry):
        pltpu.sync_copy(r0.at[pl.ds(0, _WB)], macc.at[pl.ds(pl.multiple_of(row0 + k * _WB, 8), _WB)])
        return carry

    lax.fori_loop(0, nch, zstep, 0)
    plsc.subcore_barrier()

    ccol = c * _CH
    nk = _NCHK // _NS + jnp.where(s < _NCHK % _NS, 1, 0)

    def chunk1(j):
        e0 = pl.multiple_of((s + j * _NS) * _C, 8)
        pltpu.sync_copy(mrows.at[pl.ds(e0, _C), pl.ds(ccol, _CH)], r0)
        pltpu.sync_copy(dstidx.at[pl.ds(e0, _C)], ic0)
        pltpu.sync_copy(r0, macc.at[ic0], add=True)

    def pair(jj, carry):
        j0 = 2 * jj
        e0 = pl.multiple_of((s + j0 * _NS) * _C, 8)
        e1 = pl.multiple_of((s + (j0 + 1) * _NS) * _C, 8)
        i0 = pltpu.async_copy(dstidx.at[pl.ds(e0, _C)], ic0, si0)
        i1 = pltpu.async_copy(dstidx.at[pl.ds(e1, _C)], ic1, si1)
        l0 = pltpu.async_copy(mrows.at[pl.ds(e0, _C), pl.ds(ccol, _CH)], r0, s0)
        l1 = pltpu.async_copy(mrows.at[pl.ds(e1, _C), pl.ds(ccol, _CH)], r1, s1)
        i0.wait()
        l0.wait()
        x0 = pltpu.async_copy(r0, macc.at[ic0], a0, add=True)
        i1.wait()
        l1.wait()
        x1 = pltpu.async_copy(r1, macc.at[ic1], a1, add=True)
        x0.wait()
        x1.wait()
        return carry

    lax.fori_loop(0, nk // 2, pair, 0)

    @pl.when(nk % 2 == 1)
    def _():
        chunk1(nk - 1)

    plsc.subcore_barrier()


def _seg_writeback(out, c, s, r0, macc):
    row0, nch = _own_rows(s)

    def wstep(k, carry):
        rr = pl.multiple_of(row0 + k * _WB, 8)
        pltpu.sync_copy(macc.at[pl.ds(rr, _WB)], r0.at[pl.ds(0, _WB)])
        pltpu.sync_copy(r0.at[pl.ds(0, _WB)], out.at[pl.ds(rr, _WB), pl.ds(c * _CH, _CH)])
        return carry

    lax.fori_loop(0, nch, wstep, 0)


def _segsum_body(mrows, dstidx, out, ic0, ic1, r0, r1, s0, s1, a0, a1, si0, si1, macc):
    c = lax.axis_index("c")
    s = lax.axis_index("s")
    _seg_phase1(mrows, dstidx, c, s, ic0, ic1, r0, r1, s0, s1, a0, a1, si0, si1, macc)
    _seg_writeback(out, c, s, r0, macc)


_seg_scratch = [
    pltpu.VMEM((_C,), jnp.int32),
    pltpu.VMEM((_C,), jnp.int32),
    pltpu.VMEM((_C, _CH), jnp.float32),
    pltpu.VMEM((_C, _CH), jnp.float32),
    pltpu.SemaphoreType.DMA,
    pltpu.SemaphoreType.DMA,
    pltpu.SemaphoreType.DMA,
    pltpu.SemaphoreType.DMA,
    pltpu.SemaphoreType.DMA,
    pltpu.SemaphoreType.DMA,
    pltpu.VMEM_SHARED((_N, _CH), jnp.float32),
]

_segsum = pl.kernel(
    _segsum_body,
    out_type=jax.ShapeDtypeStruct((_N, _D), jnp.float32),
    mesh=_mesh,
    scratch_types=_seg_scratch,
)


def _segsum_gather_body(mrows, dstidx, srcidx, out, gmout,
                        ic0, ic1, r0, r1, s0, s1, a0, a1, si0, si1, macc):
    c = lax.axis_index("c")
    s = lax.axis_index("s")
    _seg_phase1(mrows, dstidx, c, s, ic0, ic1, r0, r1, s0, s1, a0, a1, si0, si1, macc)

    ccol = c * _CH
    nk = _NCHK // _NS + jnp.where(s < _NCHK % _NS, 1, 0)

    def chunk1(j):
        e0 = pl.multiple_of((s + j * _NS) * _C, 8)
        pltpu.sync_copy(srcidx.at[pl.ds(e0, _C)], ic0)
        pltpu.async_copy(macc.at[ic0], r0, s0).wait()
        pltpu.sync_copy(r0, gmout.at[pl.ds(e0, _C), pl.ds(ccol, _CH)])

    def pair(jj, carry):
        j0 = 2 * jj
        e0 = pl.multiple_of((s + j0 * _NS) * _C, 8)
        e1 = pl.multiple_of((s + (j0 + 1) * _NS) * _C, 8)
        i0 = pltpu.async_copy(srcidx.at[pl.ds(e0, _C)], ic0, si0)
        i1 = pltpu.async_copy(srcidx.at[pl.ds(e1, _C)], ic1, si1)
        i0.wait()
        g0 = pltpu.async_copy(macc.at[ic0], r0, s0)
        i1.wait()
        g1 = pltpu.async_copy(macc.at[ic1], r1, s1)
        g0.wait()
        wa = pltpu.async_copy(r0, gmout.at[pl.ds(e0, _C), pl.ds(ccol, _CH)], a0)
        g1.wait()
        wb = pltpu.async_copy(r1, gmout.at[pl.ds(e1, _C), pl.ds(ccol, _CH)], a1)
        wa.wait()
        wb.wait()
        return carry

    lax.fori_loop(0, nk // 2, pair, 0)

    @pl.when(nk % 2 == 1)
    def _():
        chunk1(nk - 1)

    _seg_writeback(out, c, s, r0, macc)


_segsum_gather = pl.kernel(
    _segsum_gather_body,
    out_type=(
        jax.ShapeDtypeStruct((_N, _D), jnp.float32),
        jax.ShapeDtypeStruct((_E, _D), jnp.float32),
    ),
    mesh=_mesh,
    scratch_types=_seg_scratch,
)


def _deg_body(srcidx, out, gdout, ic0, ic1, icg0, icg1, ones, gr0, gr1, s0, s1, a0, a1, si0, si1, dacc):
    c = lax.axis_index("c")
    s = lax.axis_index("s")
    row0, nch = _own_rows(s)
    _fill(gr0, _WB, _DG, 0.0)
    _fill(ones, _C, _DG, 1.0)

    def zstep(k, carry):
        pltpu.sync_copy(gr0.at[pl.ds(0, _WB)], dacc.at[pl.ds(pl.multiple_of(row0 + k * _WB, 8), _WB)])
        return carry

    lax.fori_loop(0, nch, zstep, 0)
    plsc.subcore_barrier()

    nk = _NCHK // _NS + jnp.where(s < _NCHK % _NS, 1, 0)

    def schunk1(j):
        e0 = pl.multiple_of((s + j * _NS) * _C, 8)
        pltpu.sync_copy(srcidx.at[pl.ds(e0, _C)], ic0)
        pltpu.sync_copy(ones, dacc.at[ic0], add=True)

    def pair(jj, carry):
        j0 = 2 * jj
        e0 = pl.multiple_of((s + j0 * _NS) * _C, 8)
        e1 = pl.multiple_of((s + (j0 + 1) * _NS) * _C, 8)
        i0 = pltpu.async_copy(srcidx.at[pl.ds(e0, _C)], ic0, si0)
        i1 = pltpu.async_copy(srcidx.at[pl.ds(e1, _C)], ic1, si1)
        i0.wait()
        x0 = pltpu.async_copy(ones, dacc.at[ic0], a0, add=True)
        i1.wait()
        x1 = pltpu.async_copy(ones, dacc.at[ic1], a1, add=True)
        x0.wait()
        x1.wait()
        return carry

    lax.fori_loop(0, nk // 2, pair, 0)

    @pl.when(nk % 2 == 1)
    def _():
        schunk1(nk - 1)

    plsc.subcore_barrier()

    wid = s * _NC + c
    gnk = _NCHK // _NW + jnp.where(wid < _NCHK % _NW, 1, 0)

    def gchunk1(j):
        e0 = pl.multiple_of((wid + j * _NW) * _C, 8)
        pltpu.sync_copy(srcidx.at[pl.ds(e0, _C)], icg0)
        pltpu.async_copy(dacc.at[icg0], gr0, s0).wait()
        pltpu.sync_copy(gr0, gdout.at[pl.ds(e0, _C)])

    def gpair(jj, carry):
        j0 = 2 * jj
        e0 = pl.multiple_of((wid + j0 * _NW) * _C, 8)
        e1 = pl.multiple_of((wid + (j0 + 1) * _NW) * _C, 8)
        i0 = pltpu.async_copy(srcidx.at[pl.ds(e0, _C)], icg0, si0)
        i1 = pltpu.async_copy(srcidx.at[pl.ds(e1, _C)], icg1, si1)
        i0.wait()
        g0 = pltpu.async_copy(dacc.at[icg0], gr0, s0)
        i1.wait()
        g1 = pltpu.async_copy(dacc.at[icg1], gr1, s1)
        g0.wait()
        wa = pltpu.async_copy(gr0, gdout.at[pl.ds(e0, _C)], a0)
        g1.wait()
        wb = pltpu.async_copy(gr1, gdout.at[pl.ds(e1, _C)], a1)
        wa.wait()
        wb.wait()
        return carry

    lax.fori_loop(0, gnk // 2, gpair, 0)

    @pl.when(gnk % 2 == 1)
    def _():
        gchunk1(gnk - 1)

    @pl.when(c == 0)
    def _():
        def wstep(k, carry):
            rr = pl.multiple_of(row0 + k * _WB, 8)
            pltpu.sync_copy(dacc.at[pl.ds(rr, _WB)], gr0.at[pl.ds(0, _WB)])
            pltpu.sync_copy(gr0.at[pl.ds(0, _WB)], out.at[pl.ds(rr, _WB)])
            return carry

        lax.fori_loop(0, nch, wstep, 0)


_deg = pl.kernel(
    _deg_body,
    out_type=(
        jax.ShapeDtypeStruct((_N, _DG), jnp.float32),
        jax.ShapeDtypeStruct((_E, _DG), jnp.float32),
    ),
    mesh=_mesh,
    scratch_types=[
        pltpu.VMEM((_C,), jnp.int32),
        pltpu.VMEM((_C,), jnp.int32),
        pltpu.VMEM((_C,), jnp.int32),
        pltpu.VMEM((_C,), jnp.int32),
        pltpu.VMEM((_C, _DG), jnp.float32),
        pltpu.VMEM((_C, _DG), jnp.float32),
        pltpu.VMEM((_C, _DG), jnp.float32),
        pltpu.SemaphoreType.DMA,
        pltpu.SemaphoreType.DMA,
        pltpu.SemaphoreType.DMA,
        pltpu.SemaphoreType.DMA,
        pltpu.SemaphoreType.DMA,
        pltpu.SemaphoreType.DMA,
        pltpu.VMEM_SHARED((_N, _DG), jnp.float32),
    ],
)


_BLK_N = 400
_CN = (((1,), (1,)), ((), ()))


def _mlp_body(xb, w1, w2, wp, bpb, ob):
    xx = xb[...]
    a = lax.dot_general(xx, w1[...], _CN, preferred_element_type=jnp.float32)
    g = lax.dot_general(xx, w2[...], _CN, preferred_element_type=jnp.float32)
    hh = (a * jax.nn.sigmoid(a)) * g
    ob[...] = lax.dot_general(hh, wp[...], _CN, preferred_element_type=jnp.float32) + bpb[...]


def _mlp(x, W1, W2, Wp, bp):
    return pl.pallas_call(
        _mlp_body,
        grid=(_N // _BLK_N,),
        in_specs=[
            pl.BlockSpec((_BLK_N, _D), lambda i: (i, 0)),
            pl.BlockSpec((_HID, _D), lambda i: (0, 0)),
            pl.BlockSpec((_HID, _D), lambda i: (0, 0)),
            pl.BlockSpec((_D, _HID), lambda i: (0, 0)),
            pl.BlockSpec((1, _D), lambda i: (0, 0)),
        ],
        out_specs=pl.BlockSpec((_BLK_N, _D), lambda i: (i, 0)),
        out_shape=jax.ShapeDtypeStruct((_N, _D), jnp.float32),
    )(x, W1, W2, Wp, bp.reshape(1, _D))


def _swap(v):
    ev = (lax.broadcasted_iota(jnp.int32, v.shape, 1) % 2) == 0
    return jnp.where(ev, pltpu.roll(v, _D - 1, 1), pltpu.roll(v, 1, 1))


_BLK_E = 400
_NB_E = _E // _BLK_E
_HALF = (_E // 2) // _BLK_E


def _combine_body(gh, gm, mprev, gdeg, ca, sa, cb, sb, out):
    xe = gh[...]
    de = gdeg[...][:, :1]
    dm = gm[...] - mprev[...]
    la = ca[...] * xe + sa[...] * _swap(xe)
    lb = cb[...] * dm + sb[...] * _swap(dm)
    out[...] = jnp.where(de == 1.0, xe, la + lb / (de - 1.0 + 1e-9))


def _combine(gh, gm, mprev, gdeg, CA, SA, CB, SB):
    coef = pl.BlockSpec((1, _D), lambda i: (0, 0))
    eb = pl.BlockSpec((_BLK_E, _D), lambda i: (i, 0))
    return pl.pallas_call(
        _combine_body,
        grid=(_NB_E,),
        in_specs=[
            eb, eb,
            pl.BlockSpec((_BLK_E, _D), lambda i: ((i + _HALF) % _NB_E, 0)),
            pl.BlockSpec((_BLK_E, _DG), lambda i: (i, 0)),
            coef, coef, coef, coef,
        ],
        out_specs=eb,
        out_shape=jax.ShapeDtypeStruct((_E, _D), jnp.float32),
    )(gh, gm, mprev, gdeg, CA, SA, CB, SB)


def _final_body(xb, hb, mb, degb, ca, sa, cb, sb, out):
    hh = hb[...]
    mm = mb[...]
    dg = degb[...][:, :1]
    la = ca[...] * hh + sa[...] * _swap(hh)
    lb = cb[...] * mm + sb[...] * _swap(mm)
    val = jnp.where(dg == 0.0, hh, la + lb / (dg + 1e-9))
    out[...] = xb[...] + jnp.maximum(val, 0.0)


def _final(x, h, m, degn, CA, SA, CB, SB):
    coef = pl.BlockSpec((1, _D), lambda i: (0, 0))
    nb = pl.BlockSpec((_BLK_N, _D), lambda i: (i, 0))
    return pl.pallas_call(
        _final_body,
        grid=(_N // _BLK_N,),
        in_specs=[
            nb, nb, nb,
            pl.BlockSpec((_BLK_N, _DG), lambda i: (i, 0)),
            coef, coef, coef, coef,
        ],
        out_specs=nb,
        out_shape=jax.ShapeDtypeStruct((_N, _D), jnp.float32),
    )(x, h, m, degn, CA, SA, CB, SB)


def kernel(x, edge_index, log_dt, log_lambda_real, lambda_imag, W1, W2, Wp, bp):
    src = edge_index[0]
    dst = edge_index[1]

    dt = jnp.exp(log_dt)
    mag = jnp.exp(-jnp.exp(log_lambda_real) * dt)
    ph = lambda_imag * dt
    cw = jnp.cos(ph)
    sw = jnp.sin(ph)

    def coeffs(mx):
        cc = jnp.repeat(mx * cw, 2).reshape(1, _D)
        ss = jnp.stack((-mx * sw, mx * sw), axis=-1).reshape(1, _D)
        return cc, ss

    CA, SA = coeffs(1.0 - mag)
    CB, SB = coeffs(mag)

    h = _mlp(x, W1, W2, Wp, bp)
    degn, gdeg = _deg(src)
    gh = _gather(h, src)

    M = gh
    m, gm = _segsum_gather(gh, dst, src)
    for t in range(_T):
        M = _combine(gh, gm, M, gdeg, CA, SA, CB, SB)
        if t < _T - 1:
            m, gm = _segsum_gather(M, dst, src)
        else:
            m = _segsum(M, dst)

    return _final(x, h, m, degn, CA, SA, CB, SB)


# speedup vs baseline: 4.9487x; 1.0075x over previous
import jax
import jax.numpy as jnp
from jax import lax
from jax.experimental import pallas as pl
from jax.experimental.pallas import tpu as pltpu
from jax.experimental.pallas import tpu_sc as plsc

_N = 10000
_E = 160000
_D = 256
_HID = 512
_T = 3

_NC = 2
_NS = 16
_NW = _NC * _NS
_CH = _D // _NC
_DG = 128

_EPW = _E // _NW
_EPS = _E // _NS
_GC = 40
_SCC = 80
_C = 128
_NCHK = _E // _C
_OWN = 640
_WB = 80

_mesh = plsc.VectorSubcoreMesh(core_axis_name="c", subcore_axis_name="s")


def _fill(ref, nrows, ncols, val):
    v = jnp.full((16,), val, jnp.float32)
    per = ncols // 16

    def st(t, carry):
        ref[t // per, pl.ds((t % per) * 16, 16)] = v
        return carry

    lax.fori_loop(0, nrows * per, st, 0)


def _own_rows(s):
    row0 = s * _OWN
    nch = jnp.where(s == _NS - 1, (_N - (_NS - 1) * _OWN) // _WB, _OWN // _WB)
    return row0, nch


def _gather_body(tab, idx, out, ic0, ic1, r0, r1, s0, s1, w0, w1, si0, si1):
    c = lax.axis_index("c")
    s = lax.axis_index("s")
    wid = s * _NC + c
    nk = _NCHK // _NW + jnp.where(wid < _NCHK % _NW, 1, 0)

    def chunk1(j):
        e0 = pl.multiple_of((wid + j * _NW) * _C, 8)
        pltpu.sync_copy(idx.at[pl.ds(e0, _C)], ic0)
        pltpu.async_copy(tab.at[ic0], r0, s0).wait()
        pltpu.sync_copy(r0, out.at[pl.ds(e0, _C)])

    def pair(jj, carry):
        j0 = 2 * jj
        e0 = pl.multiple_of((wid + j0 * _NW) * _C, 8)
        e1 = pl.multiple_of((wid + (j0 + 1) * _NW) * _C, 8)
        i0 = pltpu.async_copy(idx.at[pl.ds(e0, _C)], ic0, si0)
        i1 = pltpu.async_copy(idx.at[pl.ds(e1, _C)], ic1, si1)
        i0.wait()
        g0 = pltpu.async_copy(tab.at[ic0], r0, s0)
        i1.wait()
        g1 = pltpu.async_copy(tab.at[ic1], r1, s1)
        g0.wait()
        wa = pltpu.async_copy(r0, out.at[pl.ds(e0, _C)], w0)
        g1.wait()
        wb = pltpu.async_copy(r1, out.at[pl.ds(e1, _C)], w1)
        wa.wait()
        wb.wait()
        return carry

    lax.fori_loop(0, nk // 2, pair, 0)

    @pl.when(nk % 2 == 1)
    def _():
        chunk1(nk - 1)


_gather = pl.kernel(
    _gather_body,
    out_type=jax.ShapeDtypeStruct((_E, _D), jnp.float32),
    mesh=_mesh,
    scratch_types=[
        pltpu.VMEM((_C,), jnp.int32),
        pltpu.VMEM((_C,), jnp.int32),
        pltpu.VMEM((_C, _D), jnp.float32),
        pltpu.VMEM((_C, _D), jnp.float32),
        pltpu.SemaphoreType.DMA,
        pltpu.SemaphoreType.DMA,
        pltpu.SemaphoreType.DMA,
        pltpu.SemaphoreType.DMA,
        pltpu.SemaphoreType.DMA,
        pltpu.SemaphoreType.DMA,
    ],
)


def _seg_phase1(mrows, dstidx, c, s, ic0, ic1, r0, r1, s0, s1, a0, a1, si0, si1, macc):
    row0, nch = _own_rows(s)
    _fill(r0, _WB, _CH, 0.0)

    def zstep(k, carry):
        pltpu.sync_copy(r0.at[pl.ds(0, _WB)], macc.at[pl.ds(pl.multiple_of(row0 + k * _WB, 8), _WB)])
        return carry

    lax.fori_loop(0, nch, zstep, 0)
    plsc.subcore_barrier()

    ccol = c * _CH
    nk = _NCHK // _NS + jnp.where(s < _NCHK % _NS, 1, 0)

    def chunk1(j):
        e0 = pl.multiple_of((s + j * _NS) * _C, 8)
        pltpu.sync_copy(mrows.at[pl.ds(e0, _C), pl.ds(ccol, _CH)], r0)
        pltpu.sync_copy(dstidx.at[pl.ds(e0, _C)], ic0)
        pltpu.sync_copy(r0, macc.at[ic0], add=True)

    def pair(jj, carry):
        j0 = 2 * jj
        e0 = pl.multiple_of((s + j0 * _NS) * _C, 8)
        e1 = pl.multiple_of((s + (j0 + 1) * _NS) * _C, 8)
        i0 = pltpu.async_copy(dstidx.at[pl.ds(e0, _C)], ic0, si0)
        i1 = pltpu.async_copy(dstidx.at[pl.ds(e1, _C)], ic1, si1)
        l0 = pltpu.async_copy(mrows.at[pl.ds(e0, _C), pl.ds(ccol, _CH)], r0, s0)
        l1 = pltpu.async_copy(mrows.at[pl.ds(e1, _C), pl.ds(ccol, _CH)], r1, s1)
        i0.wait()
        l0.wait()
        x0 = pltpu.async_copy(r0, macc.at[ic0], a0, add=True)
        i1.wait()
        l1.wait()
        x1 = pltpu.async_copy(r1, macc.at[ic1], a1, add=True)
        x0.wait()
        x1.wait()
        return carry

    lax.fori_loop(0, nk // 2, pair, 0)

    @pl.when(nk % 2 == 1)
    def _():
        chunk1(nk - 1)

    plsc.subcore_barrier()


def _seg_writeback(out, c, s, r0, macc):
    row0, nch = _own_rows(s)

    def wstep(k, carry):
        rr = pl.multiple_of(row0 + k * _WB, 8)
        pltpu.sync_copy(macc.at[pl.ds(rr, _WB)], r0.at[pl.ds(0, _WB)])
        pltpu.sync_copy(r0.at[pl.ds(0, _WB)], out.at[pl.ds(rr, _WB), pl.ds(c * _CH, _CH)])
        return carry

    lax.fori_loop(0, nch, wstep, 0)


def _segsum_body(mrows, dstidx, out, ic0, ic1, r0, r1, s0, s1, a0, a1, si0, si1, macc):
    c = lax.axis_index("c")
    s = lax.axis_index("s")
    _seg_phase1(mrows, dstidx, c, s, ic0, ic1, r0, r1, s0, s1, a0, a1, si0, si1, macc)
    _seg_writeback(out, c, s, r0, macc)


_seg_scratch = [
    pltpu.VMEM((_C,), jnp.int32),
    pltpu.VMEM((_C,), jnp.int32),
    pltpu.VMEM((_C, _CH), jnp.float32),
    pltpu.VMEM((_C, _CH), jnp.float32),
    pltpu.SemaphoreType.DMA,
    pltpu.SemaphoreType.DMA,
    pltpu.SemaphoreType.DMA,
    pltpu.SemaphoreType.DMA,
    pltpu.SemaphoreType.DMA,
    pltpu.SemaphoreType.DMA,
    pltpu.VMEM_SHARED((_N, _CH), jnp.float32),
]

_segsum = pl.kernel(
    _segsum_body,
    out_type=jax.ShapeDtypeStruct((_N, _D), jnp.float32),
    mesh=_mesh,
    scratch_types=_seg_scratch,
)


def _segsum_gather_body(mrows, dstidx, srcidx, gmout,
                        ic0, ic1, r0, r1, s0, s1, a0, a1, si0, si1, macc):
    c = lax.axis_index("c")
    s = lax.axis_index("s")
    _seg_phase1(mrows, dstidx, c, s, ic0, ic1, r0, r1, s0, s1, a0, a1, si0, si1, macc)

    ccol = c * _CH
    nk = _NCHK // _NS + jnp.where(s < _NCHK % _NS, 1, 0)

    def chunk1(j):
        e0 = pl.multiple_of((s + j * _NS) * _C, 8)
        pltpu.sync_copy(srcidx.at[pl.ds(e0, _C)], ic0)
        pltpu.async_copy(macc.at[ic0], r0, s0).wait()
        pltpu.sync_copy(r0, gmout.at[pl.ds(e0, _C), pl.ds(ccol, _CH)])

    def pair(jj, carry):
        j0 = 2 * jj
        e0 = pl.multiple_of((s + j0 * _NS) * _C, 8)
        e1 = pl.multiple_of((s + (j0 + 1) * _NS) * _C, 8)
        i0 = pltpu.async_copy(srcidx.at[pl.ds(e0, _C)], ic0, si0)
        i1 = pltpu.async_copy(srcidx.at[pl.ds(e1, _C)], ic1, si1)
        i0.wait()
        g0 = pltpu.async_copy(macc.at[ic0], r0, s0)
        i1.wait()
        g1 = pltpu.async_copy(macc.at[ic1], r1, s1)
        g0.wait()
        wa = pltpu.async_copy(r0, gmout.at[pl.ds(e0, _C), pl.ds(ccol, _CH)], a0)
        g1.wait()
        wb = pltpu.async_copy(r1, gmout.at[pl.ds(e1, _C), pl.ds(ccol, _CH)], a1)
        wa.wait()
        wb.wait()
        return carry

    lax.fori_loop(0, nk // 2, pair, 0)

    @pl.when(nk % 2 == 1)
    def _():
        chunk1(nk - 1)


_segsum_gather = pl.kernel(
    _segsum_gather_body,
    out_type=jax.ShapeDtypeStruct((_E, _D), jnp.float32),
    mesh=_mesh,
    scratch_types=_seg_scratch,
)


def _deg_body(srcidx, out, gdout, ic0, ic1, icg0, icg1, ones, gr0, gr1, s0, s1, a0, a1, si0, si1, dacc):
    c = lax.axis_index("c")
    s = lax.axis_index("s")
    row0, nch = _own_rows(s)
    _fill(gr0, _WB, _DG, 0.0)
    _fill(ones, _C, _DG, 1.0)

    def zstep(k, carry):
        pltpu.sync_copy(gr0.at[pl.ds(0, _WB)], dacc.at[pl.ds(pl.multiple_of(row0 + k * _WB, 8), _WB)])
        return carry

    lax.fori_loop(0, nch, zstep, 0)
    plsc.subcore_barrier()

    nk = _NCHK // _NS + jnp.where(s < _NCHK % _NS, 1, 0)

    def schunk1(j):
        e0 = pl.multiple_of((s + j * _NS) * _C, 8)
        pltpu.sync_copy(srcidx.at[pl.ds(e0, _C)], ic0)
        pltpu.sync_copy(ones, dacc.at[ic0], add=True)

    def pair(jj, carry):
        j0 = 2 * jj
        e0 = pl.multiple_of((s + j0 * _NS) * _C, 8)
        e1 = pl.multiple_of((s + (j0 + 1) * _NS) * _C, 8)
        i0 = pltpu.async_copy(srcidx.at[pl.ds(e0, _C)], ic0, si0)
        i1 = pltpu.async_copy(srcidx.at[pl.ds(e1, _C)], ic1, si1)
        i0.wait()
        x0 = pltpu.async_copy(ones, dacc.at[ic0], a0, add=True)
        i1.wait()
        x1 = pltpu.async_copy(ones, dacc.at[ic1], a1, add=True)
        x0.wait()
        x1.wait()
        return carry

    lax.fori_loop(0, nk // 2, pair, 0)

    @pl.when(nk % 2 == 1)
    def _():
        schunk1(nk - 1)

    plsc.subcore_barrier()

    wid = s * _NC + c
    gnk = _NCHK // _NW + jnp.where(wid < _NCHK % _NW, 1, 0)

    def gchunk1(j):
        e0 = pl.multiple_of((wid + j * _NW) * _C, 8)
        pltpu.sync_copy(srcidx.at[pl.ds(e0, _C)], icg0)
        pltpu.async_copy(dacc.at[icg0], gr0, s0).wait()
        pltpu.sync_copy(gr0, gdout.at[pl.ds(e0, _C)])

    def gpair(jj, carry):
        j0 = 2 * jj
        e0 = pl.multiple_of((wid + j0 * _NW) * _C, 8)
        e1 = pl.multiple_of((wid + (j0 + 1) * _NW) * _C, 8)
        i0 = pltpu.async_copy(srcidx.at[pl.ds(e0, _C)], icg0, si0)
        i1 = pltpu.async_copy(srcidx.at[pl.ds(e1, _C)], icg1, si1)
        i0.wait()
        g0 = pltpu.async_copy(dacc.at[icg0], gr0, s0)
        i1.wait()
        g1 = pltpu.async_copy(dacc.at[icg1], gr1, s1)
        g0.wait()
        wa = pltpu.async_copy(gr0, gdout.at[pl.ds(e0, _C)], a0)
        g1.wait()
        wb = pltpu.async_copy(gr1, gdout.at[pl.ds(e1, _C)], a1)
        wa.wait()
        wb.wait()
        return carry

    lax.fori_loop(0, gnk // 2, gpair, 0)

    @pl.when(gnk % 2 == 1)
    def _():
        gchunk1(gnk - 1)

    @pl.when(c == 0)
    def _():
        def wstep(k, carry):
            rr = pl.multiple_of(row0 + k * _WB, 8)
            pltpu.sync_copy(dacc.at[pl.ds(rr, _WB)], gr0.at[pl.ds(0, _WB)])
            pltpu.sync_copy(gr0.at[pl.ds(0, _WB)], out.at[pl.ds(rr, _WB)])
            return carry

        lax.fori_loop(0, nch, wstep, 0)


_deg = pl.kernel(
    _deg_body,
    out_type=(
        jax.ShapeDtypeStruct((_N, _DG), jnp.float32),
        jax.ShapeDtypeStruct((_E, _DG), jnp.float32),
    ),
    mesh=_mesh,
    scratch_types=[
        pltpu.VMEM((_C,), jnp.int32),
        pltpu.VMEM((_C,), jnp.int32),
        pltpu.VMEM((_C,), jnp.int32),
        pltpu.VMEM((_C,), jnp.int32),
        pltpu.VMEM((_C, _DG), jnp.float32),
        pltpu.VMEM((_C, _DG), jnp.float32),
        pltpu.VMEM((_C, _DG), jnp.float32),
        pltpu.SemaphoreType.DMA,
        pltpu.SemaphoreType.DMA,
        pltpu.SemaphoreType.DMA,
        pltpu.SemaphoreType.DMA,
        pltpu.SemaphoreType.DMA,
        pltpu.SemaphoreType.DMA,
        pltpu.VMEM_SHARED((_N, _DG), jnp.float32),
    ],
)


_BLK_N = 400
_CN = (((1,), (1,)), ((), ()))


def _mlp_body(xb, w1, w2, wp, bpb, ob):
    xx = xb[...]
    a = lax.dot_general(xx, w1[...], _CN, preferred_element_type=jnp.float32)
    g = lax.dot_general(xx, w2[...], _CN, preferred_element_type=jnp.float32)
    hh = (a * jax.nn.sigmoid(a)) * g
    ob[...] = lax.dot_general(hh, wp[...], _CN, preferred_element_type=jnp.float32) + bpb[...]


def _mlp(x, W1, W2, Wp, bp):
    return pl.pallas_call(
        _mlp_body,
        grid=(_N // _BLK_N,),
        in_specs=[
            pl.BlockSpec((_BLK_N, _D), lambda i: (i, 0)),
            pl.BlockSpec((_HID, _D), lambda i: (0, 0)),
            pl.BlockSpec((_HID, _D), lambda i: (0, 0)),
            pl.BlockSpec((_D, _HID), lambda i: (0, 0)),
            pl.BlockSpec((1, _D), lambda i: (0, 0)),
        ],
        out_specs=pl.BlockSpec((_BLK_N, _D), lambda i: (i, 0)),
        out_shape=jax.ShapeDtypeStruct((_N, _D), jnp.float32),
    )(x, W1, W2, Wp, bp.reshape(1, _D))


def _swap(v):
    ev = (lax.broadcasted_iota(jnp.int32, v.shape, 1) % 2) == 0
    return jnp.where(ev, pltpu.roll(v, _D - 1, 1), pltpu.roll(v, 1, 1))


_BLK_E = 400
_NB_E = _E // _BLK_E
_HALF = (_E // 2) // _BLK_E


def _combine_body(gh, gm, mprev, gdeg, ca, sa, cb, sb, out):
    xe = gh[...]
    de = gdeg[...][:, :1]
    dm = gm[...] - mprev[...]
    la = ca[...] * xe + sa[...] * _swap(xe)
    lb = cb[...] * dm + sb[...] * _swap(dm)
    out[...] = jnp.where(de == 1.0, xe, la + lb / (de - 1.0 + 1e-9))


def _combine(gh, gm, mprev, gdeg, CA, SA, CB, SB):
    coef = pl.BlockSpec((1, _D), lambda i: (0, 0))
    eb = pl.BlockSpec((_BLK_E, _D), lambda i: (i, 0))
    return pl.pallas_call(
        _combine_body,
        grid=(_NB_E,),
        in_specs=[
            eb, eb,
            pl.BlockSpec((_BLK_E, _D), lambda i: ((i + _HALF) % _NB_E, 0)),
            pl.BlockSpec((_BLK_E, _DG), lambda i: (i, 0)),
            coef, coef, coef, coef,
        ],
        out_specs=eb,
        out_shape=jax.ShapeDtypeStruct((_E, _D), jnp.float32),
    )(gh, gm, mprev, gdeg, CA, SA, CB, SB)


def _final_body(xb, hb, mb, degb, ca, sa, cb, sb, out):
    hh = hb[...]
    mm = mb[...]
    dg = degb[...][:, :1]
    la = ca[...] * hh + sa[...] * _swap(hh)
    lb = cb[...] * mm + sb[...] * _swap(mm)
    val = jnp.where(dg == 0.0, hh, la + lb / (dg + 1e-9))
    out[...] = xb[...] + jnp.maximum(val, 0.0)


def _final(x, h, m, degn, CA, SA, CB, SB):
    coef = pl.BlockSpec((1, _D), lambda i: (0, 0))
    nb = pl.BlockSpec((_BLK_N, _D), lambda i: (i, 0))
    return pl.pallas_call(
        _final_body,
        grid=(_N // _BLK_N,),
        in_specs=[
            nb, nb, nb,
            pl.BlockSpec((_BLK_N, _DG), lambda i: (i, 0)),
            coef, coef, coef, coef,
        ],
        out_specs=nb,
        out_shape=jax.ShapeDtypeStruct((_N, _D), jnp.float32),
    )(x, h, m, degn, CA, SA, CB, SB)


def kernel(x, edge_index, log_dt, log_lambda_real, lambda_imag, W1, W2, Wp, bp):
    src = edge_index[0]
    dst = edge_index[1]

    dt = jnp.exp(log_dt)
    mag = jnp.exp(-jnp.exp(log_lambda_real) * dt)
    ph = lambda_imag * dt
    cw = jnp.cos(ph)
    sw = jnp.sin(ph)

    def coeffs(mx):
        cc = jnp.repeat(mx * cw, 2).reshape(1, _D)
        ss = jnp.stack((-mx * sw, mx * sw), axis=-1).reshape(1, _D)
        return cc, ss

    CA, SA = coeffs(1.0 - mag)
    CB, SB = coeffs(mag)

    h = _mlp(x, W1, W2, Wp, bp)
    degn, gdeg = _deg(src)
    gh = _gather(h, src)

    M = gh
    gm = _segsum_gather(gh, dst, src)
    for t in range(_T):
        M = _combine(gh, gm, M, gdeg, CA, SA, CB, SB)
        if t < _T - 1:
            gm = _segsum_gather(M, dst, src)
        else:
            m = _segsum(M, dst)

    return _final(x, h, m, degn, CA, SA, CB, SB)


# speedup vs baseline: 5.5891x; 1.1294x over previous
import jax
import jax.numpy as jnp
from jax import lax
from jax.experimental import pallas as pl
from jax.experimental.pallas import tpu as pltpu
from jax.experimental.pallas import tpu_sc as plsc

_N = 10000
_E = 160000
_D = 256
_HID = 512
_T = 3

_NC = 2
_NS = 16
_NW = _NC * _NS
_CH = _D // _NC
_DG = 128

_EPW = _E // _NW
_EPS = _E // _NS
_GC = 40
_SCC = 80
_C = 128
_NCHK = _E // _C
_OWN = 640
_WB = 80

_mesh = plsc.VectorSubcoreMesh(core_axis_name="c", subcore_axis_name="s")


def _fill(ref, nrows, ncols, val):
    v = jnp.full((16,), val, jnp.float32)
    per = ncols // 16

    def st(t, carry):
        ref[t // per, pl.ds((t % per) * 16, 16)] = v
        return carry

    lax.fori_loop(0, nrows * per, st, 0)


def _own_rows(s):
    row0 = s * _OWN
    nch = jnp.where(s == _NS - 1, (_N - (_NS - 1) * _OWN) // _WB, _OWN // _WB)
    return row0, nch


def _gather_body(tab, idx, out, ic0, ic1, r0, r1, s0, s1, w0, w1, si0, si1):
    c = lax.axis_index("c")
    s = lax.axis_index("s")
    wid = s * _NC + c
    nk = _NCHK // _NW + jnp.where(wid < _NCHK % _NW, 1, 0)

    def chunk1(j):
        e0 = pl.multiple_of((wid + j * _NW) * _C, 8)
        pltpu.sync_copy(idx.at[pl.ds(e0, _C)], ic0)
        pltpu.async_copy(tab.at[ic0], r0, s0).wait()
        pltpu.sync_copy(r0, out.at[pl.ds(e0, _C)])

    def pair(jj, carry):
        j0 = 2 * jj
        e0 = pl.multiple_of((wid + j0 * _NW) * _C, 8)
        e1 = pl.multiple_of((wid + (j0 + 1) * _NW) * _C, 8)
        i0 = pltpu.async_copy(idx.at[pl.ds(e0, _C)], ic0, si0)
        i1 = pltpu.async_copy(idx.at[pl.ds(e1, _C)], ic1, si1)
        i0.wait()
        g0 = pltpu.async_copy(tab.at[ic0], r0, s0)
        i1.wait()
        g1 = pltpu.async_copy(tab.at[ic1], r1, s1)
        g0.wait()
        wa = pltpu.async_copy(r0, out.at[pl.ds(e0, _C)], w0)
        g1.wait()
        wb = pltpu.async_copy(r1, out.at[pl.ds(e1, _C)], w1)
        wa.wait()
        wb.wait()
        return carry

    lax.fori_loop(0, nk // 2, pair, 0)

    @pl.when(nk % 2 == 1)
    def _():
        chunk1(nk - 1)


_gather = pl.kernel(
    _gather_body,
    out_type=jax.ShapeDtypeStruct((_E, _D), jnp.float32),
    mesh=_mesh,
    scratch_types=[
        pltpu.VMEM((_C,), jnp.int32),
        pltpu.VMEM((_C,), jnp.int32),
        pltpu.VMEM((_C, _D), jnp.float32),
        pltpu.VMEM((_C, _D), jnp.float32),
        pltpu.SemaphoreType.DMA,
        pltpu.SemaphoreType.DMA,
        pltpu.SemaphoreType.DMA,
        pltpu.SemaphoreType.DMA,
        pltpu.SemaphoreType.DMA,
        pltpu.SemaphoreType.DMA,
    ],
)


def _seg_phase1(mrows, dstidx, c, s, ic0, ic1, r0, r1, s0, s1, a0, a1, si0, si1, macc):
    row0, nch = _own_rows(s)
    _fill(r0, _WB, _CH, 0.0)

    def zstep(k, carry):
        pltpu.sync_copy(r0.at[pl.ds(0, _WB)], macc.at[pl.ds(pl.multiple_of(row0 + k * _WB, 8), _WB)])
        return carry

    lax.fori_loop(0, nch, zstep, 0)
    plsc.subcore_barrier()

    ccol = c * _CH
    nk = _NCHK // _NS + jnp.where(s < _NCHK % _NS, 1, 0)

    def chunk1(j):
        e0 = pl.multiple_of((s + j * _NS) * _C, 8)
        pltpu.sync_copy(mrows.at[pl.ds(e0, _C), pl.ds(ccol, _CH)], r0)
        pltpu.sync_copy(dstidx.at[pl.ds(e0, _C)], ic0)
        pltpu.sync_copy(r0, macc.at[ic0], add=True)

    def pair(jj, carry):
        j0 = 2 * jj
        e0 = pl.multiple_of((s + j0 * _NS) * _C, 8)
        e1 = pl.multiple_of((s + (j0 + 1) * _NS) * _C, 8)
        i0 = pltpu.async_copy(dstidx.at[pl.ds(e0, _C)], ic0, si0)
        i1 = pltpu.async_copy(dstidx.at[pl.ds(e1, _C)], ic1, si1)
        l0 = pltpu.async_copy(mrows.at[pl.ds(e0, _C), pl.ds(ccol, _CH)], r0, s0)
        l1 = pltpu.async_copy(mrows.at[pl.ds(e1, _C), pl.ds(ccol, _CH)], r1, s1)
        i0.wait()
        l0.wait()
        x0 = pltpu.async_copy(r0, macc.at[ic0], a0, add=True)
        i1.wait()
        l1.wait()
        x1 = pltpu.async_copy(r1, macc.at[ic1], a1, add=True)
        x0.wait()
        x1.wait()
        return carry

    lax.fori_loop(0, nk // 2, pair, 0)

    @pl.when(nk % 2 == 1)
    def _():
        chunk1(nk - 1)

    plsc.subcore_barrier()


def _seg_writeback(out, c, s, r0, macc):
    row0, nch = _own_rows(s)

    def wstep(k, carry):
        rr = pl.multiple_of(row0 + k * _WB, 8)
        pltpu.sync_copy(macc.at[pl.ds(rr, _WB)], r0.at[pl.ds(0, _WB)])
        pltpu.sync_copy(r0.at[pl.ds(0, _WB)], out.at[pl.ds(rr, _WB), pl.ds(c * _CH, _CH)])
        return carry

    lax.fori_loop(0, nch, wstep, 0)


def _segsum_body(mrows, dstidx, out, ic0, ic1, r0, r1, s0, s1, a0, a1, si0, si1, macc):
    c = lax.axis_index("c")
    s = lax.axis_index("s")
    _seg_phase1(mrows, dstidx, c, s, ic0, ic1, r0, r1, s0, s1, a0, a1, si0, si1, macc)
    _seg_writeback(out, c, s, r0, macc)


_seg_scratch = [
    pltpu.VMEM((_C,), jnp.int32),
    pltpu.VMEM((_C,), jnp.int32),
    pltpu.VMEM((_C, _CH), jnp.float32),
    pltpu.VMEM((_C, _CH), jnp.float32),
    pltpu.SemaphoreType.DMA,
    pltpu.SemaphoreType.DMA,
    pltpu.SemaphoreType.DMA,
    pltpu.SemaphoreType.DMA,
    pltpu.SemaphoreType.DMA,
    pltpu.SemaphoreType.DMA,
    pltpu.VMEM_SHARED((_N, _CH), jnp.float32),
]

_segsum = pl.kernel(
    _segsum_body,
    out_type=jax.ShapeDtypeStruct((_N, _D), jnp.float32),
    mesh=_mesh,
    scratch_types=_seg_scratch,
)


def _segsum_gather_body(mrows, dstidx, srcidx, gmout,
                        ic0, ic1, r0, r1, s0, s1, a0, a1, si0, si1, macc):
    c = lax.axis_index("c")
    s = lax.axis_index("s")
    _seg_phase1(mrows, dstidx, c, s, ic0, ic1, r0, r1, s0, s1, a0, a1, si0, si1, macc)

    ccol = c * _CH
    nk = _NCHK // _NS + jnp.where(s < _NCHK % _NS, 1, 0)

    def chunk1(j):
        e0 = pl.multiple_of((s + j * _NS) * _C, 8)
        pltpu.sync_copy(srcidx.at[pl.ds(e0, _C)], ic0)
        pltpu.async_copy(macc.at[ic0], r0, s0).wait()
        pltpu.sync_copy(r0, gmout.at[pl.ds(e0, _C), pl.ds(ccol, _CH)])

    def pair(jj, carry):
        j0 = 2 * jj
        e0 = pl.multiple_of((s + j0 * _NS) * _C, 8)
        e1 = pl.multiple_of((s + (j0 + 1) * _NS) * _C, 8)
        i0 = pltpu.async_copy(srcidx.at[pl.ds(e0, _C)], ic0, si0)
        i1 = pltpu.async_copy(srcidx.at[pl.ds(e1, _C)], ic1, si1)
        i0.wait()
        g0 = pltpu.async_copy(macc.at[ic0], r0, s0)
        i1.wait()
        g1 = pltpu.async_copy(macc.at[ic1], r1, s1)
        g0.wait()
        wa = pltpu.async_copy(r0, gmout.at[pl.ds(e0, _C), pl.ds(ccol, _CH)], a0)
        g1.wait()
        wb = pltpu.async_copy(r1, gmout.at[pl.ds(e1, _C), pl.ds(ccol, _CH)], a1)
        wa.wait()
        wb.wait()
        return carry

    lax.fori_loop(0, nk // 2, pair, 0)

    @pl.when(nk % 2 == 1)
    def _():
        chunk1(nk - 1)


_segsum_gather = pl.kernel(
    _segsum_gather_body,
    out_type=jax.ShapeDtypeStruct((_E, _D), jnp.float32),
    mesh=_mesh,
    scratch_types=_seg_scratch,
)


def _deg_body(srcidx, out, gdout, ic0, ic1, icg0, icg1, ones, gr0, gr1, s0, s1, a0, a1, si0, si1, dacc):
    c = lax.axis_index("c")
    s = lax.axis_index("s")
    row0, nch = _own_rows(s)
    _fill(gr0, _WB, _DG, 0.0)
    _fill(ones, _C, _DG, 1.0)

    def zstep(k, carry):
        pltpu.sync_copy(gr0.at[pl.ds(0, _WB)], dacc.at[pl.ds(pl.multiple_of(row0 + k * _WB, 8), _WB)])
        return carry

    lax.fori_loop(0, nch, zstep, 0)
    plsc.subcore_barrier()

    nk = _NCHK // _NS + jnp.where(s < _NCHK % _NS, 1, 0)

    def schunk1(j):
        e0 = pl.multiple_of((s + j * _NS) * _C, 8)
        pltpu.sync_copy(srcidx.at[pl.ds(e0, _C)], ic0)
        pltpu.sync_copy(ones, dacc.at[ic0], add=True)

    def pair(jj, carry):
        j0 = 2 * jj
        e0 = pl.multiple_of((s + j0 * _NS) * _C, 8)
        e1 = pl.multiple_of((s + (j0 + 1) * _NS) * _C, 8)
        i0 = pltpu.async_copy(srcidx.at[pl.ds(e0, _C)], ic0, si0)
        i1 = pltpu.async_copy(srcidx.at[pl.ds(e1, _C)], ic1, si1)
        i0.wait()
        x0 = pltpu.async_copy(ones, dacc.at[ic0], a0, add=True)
        i1.wait()
        x1 = pltpu.async_copy(ones, dacc.at[ic1], a1, add=True)
        x0.wait()
        x1.wait()
        return carry

    lax.fori_loop(0, nk // 2, pair, 0)

    @pl.when(nk % 2 == 1)
    def _():
        schunk1(nk - 1)

    plsc.subcore_barrier()

    wid = s * _NC + c
    gnk = _NCHK // _NW + jnp.where(wid < _NCHK % _NW, 1, 0)

    def gchunk1(j):
        e0 = pl.multiple_of((wid + j * _NW) * _C, 8)
        pltpu.sync_copy(srcidx.at[pl.ds(e0, _C)], icg0)
        pltpu.async_copy(dacc.at[icg0], gr0, s0).wait()
        pltpu.sync_copy(gr0, gdout.at[pl.ds(e0, _C)])

    def gpair(jj, carry):
        j0 = 2 * jj
        e0 = pl.multiple_of((wid + j0 * _NW) * _C, 8)
        e1 = pl.multiple_of((wid + (j0 + 1) * _NW) * _C, 8)
        i0 = pltpu.async_copy(srcidx.at[pl.ds(e0, _C)], icg0, si0)
        i1 = pltpu.async_copy(srcidx.at[pl.ds(e1, _C)], icg1, si1)
        i0.wait()
        g0 = pltpu.async_copy(dacc.at[icg0], gr0, s0)
        i1.wait()
        g1 = pltpu.async_copy(dacc.at[icg1], gr1, s1)
        g0.wait()
        wa = pltpu.async_copy(gr0, gdout.at[pl.ds(e0, _C)], a0)
        g1.wait()
        wb = pltpu.async_copy(gr1, gdout.at[pl.ds(e1, _C)], a1)
        wa.wait()
        wb.wait()
        return carry

    lax.fori_loop(0, gnk // 2, gpair, 0)

    @pl.when(gnk % 2 == 1)
    def _():
        gchunk1(gnk - 1)

    @pl.when(c == 0)
    def _():
        def wstep(k, carry):
            rr = pl.multiple_of(row0 + k * _WB, 8)
            pltpu.sync_copy(dacc.at[pl.ds(rr, _WB)], gr0.at[pl.ds(0, _WB)])
            pltpu.sync_copy(gr0.at[pl.ds(0, _WB)], out.at[pl.ds(rr, _WB)])
            return carry

        lax.fori_loop(0, nch, wstep, 0)


_deg = pl.kernel(
    _deg_body,
    out_type=(
        jax.ShapeDtypeStruct((_N, _DG), jnp.float32),
        jax.ShapeDtypeStruct((_E, _DG), jnp.float32),
    ),
    mesh=_mesh,
    scratch_types=[
        pltpu.VMEM((_C,), jnp.int32),
        pltpu.VMEM((_C,), jnp.int32),
        pltpu.VMEM((_C,), jnp.int32),
        pltpu.VMEM((_C,), jnp.int32),
        pltpu.VMEM((_C, _DG), jnp.float32),
        pltpu.VMEM((_C, _DG), jnp.float32),
        pltpu.VMEM((_C, _DG), jnp.float32),
        pltpu.SemaphoreType.DMA,
        pltpu.SemaphoreType.DMA,
        pltpu.SemaphoreType.DMA,
        pltpu.SemaphoreType.DMA,
        pltpu.SemaphoreType.DMA,
        pltpu.SemaphoreType.DMA,
        pltpu.VMEM_SHARED((_N, _DG), jnp.float32),
    ],
)


_BLK_N = 400
_CN = (((1,), (1,)), ((), ()))


def _mlp_body(xb, w1, w2, wp, bpb, ob):
    xx = xb[...]
    a = lax.dot_general(xx, w1[...], _CN, preferred_element_type=jnp.float32)
    g = lax.dot_general(xx, w2[...], _CN, preferred_element_type=jnp.float32)
    hh = (a * jax.nn.sigmoid(a)) * g
    ob[...] = lax.dot_general(hh, wp[...], _CN, preferred_element_type=jnp.float32) + bpb[...]


def _mlp(x, W1, W2, Wp, bp):
    return pl.pallas_call(
        _mlp_body,
        grid=(_N // _BLK_N,),
        in_specs=[
            pl.BlockSpec((_BLK_N, _D), lambda i: (i, 0)),
            pl.BlockSpec((_HID, _D), lambda i: (0, 0)),
            pl.BlockSpec((_HID, _D), lambda i: (0, 0)),
            pl.BlockSpec((_D, _HID), lambda i: (0, 0)),
            pl.BlockSpec((1, _D), lambda i: (0, 0)),
        ],
        out_specs=pl.BlockSpec((_BLK_N, _D), lambda i: (i, 0)),
        out_shape=jax.ShapeDtypeStruct((_N, _D), jnp.float32),
    )(x, W1, W2, Wp, bp.reshape(1, _D))


def _swap(v):
    ev = (lax.broadcasted_iota(jnp.int32, v.shape, 1) % 2) == 0
    return jnp.where(ev, pltpu.roll(v, _D - 1, 1), pltpu.roll(v, 1, 1))


_BLK_E = 800
_NB_E = _E // _BLK_E
_HALF = (_E // 2) // _BLK_E


def _combine_body(gh, gm, mprev, gdeg, ca, sa, cb, sb, out):
    xe = gh[...]
    de = gdeg[...][:, :1]
    dm = gm[...] - mprev[...]
    la = ca[...] * xe + sa[...] * _swap(xe)
    lb = cb[...] * dm + sb[...] * _swap(dm)
    out[...] = jnp.where(de == 1.0, xe, la + lb / (de - 1.0 + 1e-9))


def _combine(gh, gm, mprev, gdeg, CA, SA, CB, SB):
    coef = pl.BlockSpec((1, _D), lambda i: (0, 0))
    eb = pl.BlockSpec((_BLK_E, _D), lambda i: (i, 0))
    return pl.pallas_call(
        _combine_body,
        grid=(_NB_E,),
        in_specs=[
            eb, eb,
            pl.BlockSpec((_BLK_E, _D), lambda i: ((i + _HALF) % _NB_E, 0)),
            pl.BlockSpec((_BLK_E, _DG), lambda i: (i, 0)),
            coef, coef, coef, coef,
        ],
        out_specs=eb,
        out_shape=jax.ShapeDtypeStruct((_E, _D), jnp.float32),
    )(gh, gm, mprev, gdeg, CA, SA, CB, SB)


def _final_body(xb, hb, mb, degb, ca, sa, cb, sb, out):
    hh = hb[...]
    mm = mb[...]
    dg = degb[...][:, :1]
    la = ca[...] * hh + sa[...] * _swap(hh)
    lb = cb[...] * mm + sb[...] * _swap(mm)
    val = jnp.where(dg == 0.0, hh, la + lb / (dg + 1e-9))
    out[...] = xb[...] + jnp.maximum(val, 0.0)


def _final(x, h, m, degn, CA, SA, CB, SB):
    coef = pl.BlockSpec((1, _D), lambda i: (0, 0))
    nb = pl.BlockSpec((_BLK_N, _D), lambda i: (i, 0))
    return pl.pallas_call(
        _final_body,
        grid=(_N // _BLK_N,),
        in_specs=[
            nb, nb, nb,
            pl.BlockSpec((_BLK_N, _DG), lambda i: (i, 0)),
            coef, coef, coef, coef,
        ],
        out_specs=nb,
        out_shape=jax.ShapeDtypeStruct((_N, _D), jnp.float32),
    )(x, h, m, degn, CA, SA, CB, SB)


def kernel(x, edge_index, log_dt, log_lambda_real, lambda_imag, W1, W2, Wp, bp):
    src = edge_index[0]
    dst = edge_index[1]

    dt = jnp.exp(log_dt)
    mag = jnp.exp(-jnp.exp(log_lambda_real) * dt)
    ph = lambda_imag * dt
    cw = jnp.cos(ph)
    sw = jnp.sin(ph)

    def coeffs(mx):
        cc = jnp.repeat(mx * cw, 2).reshape(1, _D)
        ss = jnp.stack((-mx * sw, mx * sw), axis=-1).reshape(1, _D)
        return cc, ss

    CA, SA = coeffs(1.0 - mag)
    CB, SB = coeffs(mag)

    h = _mlp(x, W1, W2, Wp, bp)
    degn, gdeg = _deg(src)
    gh = _gather(h, src)

    M = gh
    gm = _segsum_gather(gh, dst, src)
    for t in range(_T):
        M = _combine(gh, gm, M, gdeg, CA, SA, CB, SB)
        if t < _T - 1:
            gm = _segsum_gather(M, dst, src)
        else:
            m = _segsum(M, dst)

    return _final(x, h, m, degn, CA, SA, CB, SB)


# speedup vs baseline: 6.0727x; 1.0865x over previous
import jax
import jax.numpy as jnp
from jax import lax
from jax.experimental import pallas as pl
from jax.experimental.pallas import tpu as pltpu
from jax.experimental.pallas import tpu_sc as plsc

_N = 10000
_E = 160000
_D = 256
_HID = 512
_T = 3

_NC = 2
_NS = 16
_NW = _NC * _NS
_CH = _D // _NC
_DG = 128

_EPW = _E // _NW
_EPS = _E // _NS
_GC = 40
_SCC = 80
_C = 128
_NCHK = _E // _C
_OWN = 640
_WB = 80

_mesh = plsc.VectorSubcoreMesh(core_axis_name="c", subcore_axis_name="s")


def _fill(ref, nrows, ncols, val):
    v = jnp.full((16,), val, jnp.float32)
    per = ncols // 16

    def st(t, carry):
        ref[t // per, pl.ds((t % per) * 16, 16)] = v
        return carry

    lax.fori_loop(0, nrows * per, st, 0)


def _own_rows(s):
    row0 = s * _OWN
    nch = jnp.where(s == _NS - 1, (_N - (_NS - 1) * _OWN) // _WB, _OWN // _WB)
    return row0, nch


def _gather_body(tab, idx, out, ic0, ic1, r0, r1, s0, s1, w0, w1, si0, si1):
    c = lax.axis_index("c")
    s = lax.axis_index("s")
    wid = s * _NC + c
    nk = _NCHK // _NW + jnp.where(wid < _NCHK % _NW, 1, 0)

    def chunk1(j):
        e0 = pl.multiple_of((wid + j * _NW) * _C, 8)
        pltpu.sync_copy(idx.at[pl.ds(e0, _C)], ic0)
        pltpu.async_copy(tab.at[ic0], r0, s0).wait()
        pltpu.sync_copy(r0, out.at[pl.ds(e0, _C)])

    def pair(jj, carry):
        j0 = 2 * jj
        e0 = pl.multiple_of((wid + j0 * _NW) * _C, 8)
        e1 = pl.multiple_of((wid + (j0 + 1) * _NW) * _C, 8)
        i0 = pltpu.async_copy(idx.at[pl.ds(e0, _C)], ic0, si0)
        i1 = pltpu.async_copy(idx.at[pl.ds(e1, _C)], ic1, si1)
        i0.wait()
        g0 = pltpu.async_copy(tab.at[ic0], r0, s0)
        i1.wait()
        g1 = pltpu.async_copy(tab.at[ic1], r1, s1)
        g0.wait()
        wa = pltpu.async_copy(r0, out.at[pl.ds(e0, _C)], w0)
        g1.wait()
        wb = pltpu.async_copy(r1, out.at[pl.ds(e1, _C)], w1)
        wa.wait()
        wb.wait()
        return carry

    lax.fori_loop(0, nk // 2, pair, 0)

    @pl.when(nk % 2 == 1)
    def _():
        chunk1(nk - 1)


_gather = pl.kernel(
    _gather_body,
    out_type=jax.ShapeDtypeStruct((_E, _D), jnp.float32),
    mesh=_mesh,
    scratch_types=[
        pltpu.VMEM((_C,), jnp.int32),
        pltpu.VMEM((_C,), jnp.int32),
        pltpu.VMEM((_C, _D), jnp.float32),
        pltpu.VMEM((_C, _D), jnp.float32),
        pltpu.SemaphoreType.DMA,
        pltpu.SemaphoreType.DMA,
        pltpu.SemaphoreType.DMA,
        pltpu.SemaphoreType.DMA,
        pltpu.SemaphoreType.DMA,
        pltpu.SemaphoreType.DMA,
    ],
)


def _seg_phase1(mrows, dstidx, c, s, ic0, ic1, r0, r1, s0, s1, a0, a1, si0, si1, macc):
    row0, nch = _own_rows(s)
    _fill(r0, _WB, _CH, 0.0)

    def zstep(k, carry):
        pltpu.sync_copy(r0.at[pl.ds(0, _WB)], macc.at[pl.ds(pl.multiple_of(row0 + k * _WB, 8), _WB)])
        return carry

    lax.fori_loop(0, nch, zstep, 0)
    plsc.subcore_barrier()

    ccol = c * _CH
    nk = _NCHK // _NS + jnp.where(s < _NCHK % _NS, 1, 0)

    def chunk1(j):
        e0 = pl.multiple_of((s + j * _NS) * _C, 8)
        pltpu.sync_copy(mrows.at[pl.ds(e0, _C), pl.ds(ccol, _CH)], r0)
        pltpu.sync_copy(dstidx.at[pl.ds(e0, _C)], ic0)
        pltpu.sync_copy(r0, macc.at[ic0], add=True)

    def pair(jj, carry):
        j0 = 2 * jj
        e0 = pl.multiple_of((s + j0 * _NS) * _C, 8)
        e1 = pl.multiple_of((s + (j0 + 1) * _NS) * _C, 8)
        i0 = pltpu.async_copy(dstidx.at[pl.ds(e0, _C)], ic0, si0)
        i1 = pltpu.async_copy(dstidx.at[pl.ds(e1, _C)], ic1, si1)
        l0 = pltpu.async_copy(mrows.at[pl.ds(e0, _C), pl.ds(ccol, _CH)], r0, s0)
        l1 = pltpu.async_copy(mrows.at[pl.ds(e1, _C), pl.ds(ccol, _CH)], r1, s1)
        i0.wait()
        l0.wait()
        x0 = pltpu.async_copy(r0, macc.at[ic0], a0, add=True)
        i1.wait()
        l1.wait()
        x1 = pltpu.async_copy(r1, macc.at[ic1], a1, add=True)
        x0.wait()
        x1.wait()
        return carry

    lax.fori_loop(0, nk // 2, pair, 0)

    @pl.when(nk % 2 == 1)
    def _():
        chunk1(nk - 1)

    plsc.subcore_barrier()


def _seg_writeback(out, c, s, r0, macc):
    row0, nch = _own_rows(s)

    def wstep(k, carry):
        rr = pl.multiple_of(row0 + k * _WB, 8)
        pltpu.sync_copy(macc.at[pl.ds(rr, _WB)], r0.at[pl.ds(0, _WB)])
        pltpu.sync_copy(r0.at[pl.ds(0, _WB)], out.at[pl.ds(rr, _WB), pl.ds(c * _CH, _CH)])
        return carry

    lax.fori_loop(0, nch, wstep, 0)


def _segsum_body(mrows, dstidx, out, ic0, ic1, r0, r1, s0, s1, a0, a1, si0, si1, macc):
    c = lax.axis_index("c")
    s = lax.axis_index("s")
    _seg_phase1(mrows, dstidx, c, s, ic0, ic1, r0, r1, s0, s1, a0, a1, si0, si1, macc)
    _seg_writeback(out, c, s, r0, macc)


_seg_scratch = [
    pltpu.VMEM((_C,), jnp.int32),
    pltpu.VMEM((_C,), jnp.int32),
    pltpu.VMEM((_C, _CH), jnp.float32),
    pltpu.VMEM((_C, _CH), jnp.float32),
    pltpu.SemaphoreType.DMA,
    pltpu.SemaphoreType.DMA,
    pltpu.SemaphoreType.DMA,
    pltpu.SemaphoreType.DMA,
    pltpu.SemaphoreType.DMA,
    pltpu.SemaphoreType.DMA,
    pltpu.VMEM_SHARED((_N, _CH), jnp.float32),
]

_segsum = pl.kernel(
    _segsum_body,
    out_type=jax.ShapeDtypeStruct((_N, _D), jnp.float32),
    mesh=_mesh,
    scratch_types=_seg_scratch,
)


def _segsum_gather_body(mrows, dstidx, srcidx, gmout,
                        ic0, ic1, r0, r1, s0, s1, a0, a1, si0, si1, macc):
    c = lax.axis_index("c")
    s = lax.axis_index("s")
    _seg_phase1(mrows, dstidx, c, s, ic0, ic1, r0, r1, s0, s1, a0, a1, si0, si1, macc)

    ccol = c * _CH
    nk = _NCHK // _NS + jnp.where(s < _NCHK % _NS, 1, 0)

    def chunk1(j):
        e0 = pl.multiple_of((s + j * _NS) * _C, 8)
        pltpu.sync_copy(srcidx.at[pl.ds(e0, _C)], ic0)
        pltpu.async_copy(macc.at[ic0], r0, s0).wait()
        pltpu.sync_copy(r0, gmout.at[pl.ds(e0, _C), pl.ds(ccol, _CH)])

    def pair(jj, carry):
        j0 = 2 * jj
        e0 = pl.multiple_of((s + j0 * _NS) * _C, 8)
        e1 = pl.multiple_of((s + (j0 + 1) * _NS) * _C, 8)
        i0 = pltpu.async_copy(srcidx.at[pl.ds(e0, _C)], ic0, si0)
        i1 = pltpu.async_copy(srcidx.at[pl.ds(e1, _C)], ic1, si1)
        i0.wait()
        g0 = pltpu.async_copy(macc.at[ic0], r0, s0)
        i1.wait()
        g1 = pltpu.async_copy(macc.at[ic1], r1, s1)
        g0.wait()
        wa = pltpu.async_copy(r0, gmout.at[pl.ds(e0, _C), pl.ds(ccol, _CH)], a0)
        g1.wait()
        wb = pltpu.async_copy(r1, gmout.at[pl.ds(e1, _C), pl.ds(ccol, _CH)], a1)
        wa.wait()
        wb.wait()
        return carry

    lax.fori_loop(0, nk // 2, pair, 0)

    @pl.when(nk % 2 == 1)
    def _():
        chunk1(nk - 1)


_segsum_gather = pl.kernel(
    _segsum_gather_body,
    out_type=jax.ShapeDtypeStruct((_E, _D), jnp.float32),
    mesh=_mesh,
    scratch_types=_seg_scratch,
)


def _deg_body(srcidx, out, gdout, ic0, ic1, icg0, icg1, ones, gr0, gr1, s0, s1, a0, a1, si0, si1, dacc):
    c = lax.axis_index("c")
    s = lax.axis_index("s")
    row0, nch = _own_rows(s)
    _fill(gr0, _WB, _DG, 0.0)
    _fill(ones, _C, _DG, 1.0)

    def zstep(k, carry):
        pltpu.sync_copy(gr0.at[pl.ds(0, _WB)], dacc.at[pl.ds(pl.multiple_of(row0 + k * _WB, 8), _WB)])
        return carry

    lax.fori_loop(0, nch, zstep, 0)
    plsc.subcore_barrier()

    nk = _NCHK // _NS + jnp.where(s < _NCHK % _NS, 1, 0)

    def schunk1(j):
        e0 = pl.multiple_of((s + j * _NS) * _C, 8)
        pltpu.sync_copy(srcidx.at[pl.ds(e0, _C)], ic0)
        pltpu.sync_copy(ones, dacc.at[ic0], add=True)

    def pair(jj, carry):
        j0 = 2 * jj
        e0 = pl.multiple_of((s + j0 * _NS) * _C, 8)
        e1 = pl.multiple_of((s + (j0 + 1) * _NS) * _C, 8)
        i0 = pltpu.async_copy(srcidx.at[pl.ds(e0, _C)], ic0, si0)
        i1 = pltpu.async_copy(srcidx.at[pl.ds(e1, _C)], ic1, si1)
        i0.wait()
        x0 = pltpu.async_copy(ones, dacc.at[ic0], a0, add=True)
        i1.wait()
        x1 = pltpu.async_copy(ones, dacc.at[ic1], a1, add=True)
        x0.wait()
        x1.wait()
        return carry

    lax.fori_loop(0, nk // 2, pair, 0)

    @pl.when(nk % 2 == 1)
    def _():
        schunk1(nk - 1)

    plsc.subcore_barrier()

    wid = s * _NC + c
    gnk = _NCHK // _NW + jnp.where(wid < _NCHK % _NW, 1, 0)

    def gchunk1(j):
        e0 = pl.multiple_of((wid + j * _NW) * _C, 8)
        pltpu.sync_copy(srcidx.at[pl.ds(e0, _C)], icg0)
        pltpu.async_copy(dacc.at[icg0], gr0, s0).wait()
        pltpu.sync_copy(gr0, gdout.at[pl.ds(e0, _C)])

    def gpair(jj, carry):
        j0 = 2 * jj
        e0 = pl.multiple_of((wid + j0 * _NW) * _C, 8)
        e1 = pl.multiple_of((wid + (j0 + 1) * _NW) * _C, 8)
        i0 = pltpu.async_copy(srcidx.at[pl.ds(e0, _C)], icg0, si0)
        i1 = pltpu.async_copy(srcidx.at[pl.ds(e1, _C)], icg1, si1)
        i0.wait()
        g0 = pltpu.async_copy(dacc.at[icg0], gr0, s0)
        i1.wait()
        g1 = pltpu.async_copy(dacc.at[icg1], gr1, s1)
        g0.wait()
        wa = pltpu.async_copy(gr0, gdout.at[pl.ds(e0, _C)], a0)
        g1.wait()
        wb = pltpu.async_copy(gr1, gdout.at[pl.ds(e1, _C)], a1)
        wa.wait()
        wb.wait()
        return carry

    lax.fori_loop(0, gnk // 2, gpair, 0)

    @pl.when(gnk % 2 == 1)
    def _():
        gchunk1(gnk - 1)

    @pl.when(c == 0)
    def _():
        def wstep(k, carry):
            rr = pl.multiple_of(row0 + k * _WB, 8)
            pltpu.sync_copy(dacc.at[pl.ds(rr, _WB)], gr0.at[pl.ds(0, _WB)])
            pltpu.sync_copy(gr0.at[pl.ds(0, _WB)], out.at[pl.ds(rr, _WB)])
            return carry

        lax.fori_loop(0, nch, wstep, 0)


_deg = pl.kernel(
    _deg_body,
    out_type=(
        jax.ShapeDtypeStruct((_N, _DG), jnp.float32),
        jax.ShapeDtypeStruct((_E, _DG), jnp.float32),
    ),
    mesh=_mesh,
    scratch_types=[
        pltpu.VMEM((_C,), jnp.int32),
        pltpu.VMEM((_C,), jnp.int32),
        pltpu.VMEM((_C,), jnp.int32),
        pltpu.VMEM((_C,), jnp.int32),
        pltpu.VMEM((_C, _DG), jnp.float32),
        pltpu.VMEM((_C, _DG), jnp.float32),
        pltpu.VMEM((_C, _DG), jnp.float32),
        pltpu.SemaphoreType.DMA,
        pltpu.SemaphoreType.DMA,
        pltpu.SemaphoreType.DMA,
        pltpu.SemaphoreType.DMA,
        pltpu.SemaphoreType.DMA,
        pltpu.SemaphoreType.DMA,
        pltpu.VMEM_SHARED((_N, _DG), jnp.float32),
    ],
)


_BLK_N = 1000
_CN = (((1,), (1,)), ((), ()))


def _mlp_body(xb, w1, w2, wp, bpb, ob):
    xx = xb[...]
    a = lax.dot_general(xx, w1[...], _CN, preferred_element_type=jnp.float32)
    g = lax.dot_general(xx, w2[...], _CN, preferred_element_type=jnp.float32)
    hh = (a * jax.nn.sigmoid(a)) * g
    ob[...] = lax.dot_general(hh, wp[...], _CN, preferred_element_type=jnp.float32) + bpb[...]


def _mlp(x, W1, W2, Wp, bp):
    return pl.pallas_call(
        _mlp_body,
        grid=(_N // _BLK_N,),
        in_specs=[
            pl.BlockSpec((_BLK_N, _D), lambda i: (i, 0)),
            pl.BlockSpec((_HID, _D), lambda i: (0, 0)),
            pl.BlockSpec((_HID, _D), lambda i: (0, 0)),
            pl.BlockSpec((_D, _HID), lambda i: (0, 0)),
            pl.BlockSpec((1, _D), lambda i: (0, 0)),
        ],
        out_specs=pl.BlockSpec((_BLK_N, _D), lambda i: (i, 0)),
        out_shape=jax.ShapeDtypeStruct((_N, _D), jnp.float32),
    )(x, W1, W2, Wp, bp.reshape(1, _D))


def _swap(v):
    ev = (lax.broadcasted_iota(jnp.int32, v.shape, 1) % 2) == 0
    return jnp.where(ev, pltpu.roll(v, _D - 1, 1), pltpu.roll(v, 1, 1))


_BLK_E = 1600
_NB_E = _E // _BLK_E
_HALF = (_E // 2) // _BLK_E


def _combine_body(gh, gm, mprev, gdeg, ca, sa, cb, sb, out):
    xe = gh[...]
    de = gdeg[...][:, :1]
    dm = gm[...] - mprev[...]
    la = ca[...] * xe + sa[...] * _swap(xe)
    lb = cb[...] * dm + sb[...] * _swap(dm)
    out[...] = jnp.where(de == 1.0, xe, la + lb / (de - 1.0 + 1e-9))


def _combine(gh, gm, mprev, gdeg, CA, SA, CB, SB):
    coef = pl.BlockSpec((1, _D), lambda i: (0, 0))
    eb = pl.BlockSpec((_BLK_E, _D), lambda i: (i, 0))
    return pl.pallas_call(
        _combine_body,
        grid=(_NB_E,),
        in_specs=[
            eb, eb,
            pl.BlockSpec((_BLK_E, _D), lambda i: ((i + _HALF) % _NB_E, 0)),
            pl.BlockSpec((_BLK_E, _DG), lambda i: (i, 0)),
            coef, coef, coef, coef,
        ],
        out_specs=eb,
        out_shape=jax.ShapeDtypeStruct((_E, _D), jnp.float32),
    )(gh, gm, mprev, gdeg, CA, SA, CB, SB)


def _final_body(xb, hb, mb, degb, ca, sa, cb, sb, out):
    hh = hb[...]
    mm = mb[...]
    dg = degb[...][:, :1]
    la = ca[...] * hh + sa[...] * _swap(hh)
    lb = cb[...] * mm + sb[...] * _swap(mm)
    val = jnp.where(dg == 0.0, hh, la + lb / (dg + 1e-9))
    out[...] = xb[...] + jnp.maximum(val, 0.0)


def _final(x, h, m, degn, CA, SA, CB, SB):
    coef = pl.BlockSpec((1, _D), lambda i: (0, 0))
    nb = pl.BlockSpec((_BLK_N, _D), lambda i: (i, 0))
    return pl.pallas_call(
        _final_body,
        grid=(_N // _BLK_N,),
        in_specs=[
            nb, nb, nb,
            pl.BlockSpec((_BLK_N, _DG), lambda i: (i, 0)),
            coef, coef, coef, coef,
        ],
        out_specs=nb,
        out_shape=jax.ShapeDtypeStruct((_N, _D), jnp.float32),
    )(x, h, m, degn, CA, SA, CB, SB)


def kernel(x, edge_index, log_dt, log_lambda_real, lambda_imag, W1, W2, Wp, bp):
    src = edge_index[0]
    dst = edge_index[1]

    dt = jnp.exp(log_dt)
    mag = jnp.exp(-jnp.exp(log_lambda_real) * dt)
    ph = lambda_imag * dt
    cw = jnp.cos(ph)
    sw = jnp.sin(ph)

    def coeffs(mx):
        cc = jnp.repeat(mx * cw, 2).reshape(1, _D)
        ss = jnp.stack((-mx * sw, mx * sw), axis=-1).reshape(1, _D)
        return cc, ss

    CA, SA = coeffs(1.0 - mag)
    CB, SB = coeffs(mag)

    h = _mlp(x, W1, W2, Wp, bp)
    degn, gdeg = _deg(src)
    gh = _gather(h, src)

    M = gh
    gm = _segsum_gather(gh, dst, src)
    for t in range(_T):
        M = _combine(gh, gm, M, gdeg, CA, SA, CB, SB)
        if t < _T - 1:
            gm = _segsum_gather(M, dst, src)
        else:
            m = _segsum(M, dst)

    return _final(x, h, m, degn, CA, SA, CB, SB)


# speedup vs baseline: 6.2890x; 1.0356x over previous
import jax
import jax.numpy as jnp
from jax import lax
from jax.experimental import pallas as pl
from jax.experimental.pallas import tpu as pltpu
from jax.experimental.pallas import tpu_sc as plsc

_N = 10000
_E = 160000
_D = 256
_HID = 512
_T = 3

_NC = 2
_NS = 16
_NW = _NC * _NS
_CH = _D // _NC
_DG = 128

_EPW = _E // _NW
_EPS = _E // _NS
_GC = 40
_SCC = 80
_C = 128
_NCHK = _E // _C
_OWN = 640
_WB = 80

_mesh = plsc.VectorSubcoreMesh(core_axis_name="c", subcore_axis_name="s")


def _fill(ref, nrows, ncols, val):
    v = jnp.full((16,), val, jnp.float32)
    per = ncols // 16

    def st(t, carry):
        ref[t // per, pl.ds((t % per) * 16, 16)] = v
        return carry

    lax.fori_loop(0, nrows * per, st, 0)


def _own_rows(s):
    row0 = s * _OWN
    nch = jnp.where(s == _NS - 1, (_N - (_NS - 1) * _OWN) // _WB, _OWN // _WB)
    return row0, nch


def _gather_body(tab, idx, out, ic0, ic1, r0, r1, s0, s1, w0, w1, si0, si1):
    c = lax.axis_index("c")
    s = lax.axis_index("s")
    wid = s * _NC + c
    nk = _NCHK // _NW + jnp.where(wid < _NCHK % _NW, 1, 0)

    def chunk1(j):
        e0 = pl.multiple_of((wid + j * _NW) * _C, 8)
        pltpu.sync_copy(idx.at[pl.ds(e0, _C)], ic0)
        pltpu.async_copy(tab.at[ic0], r0, s0).wait()
        pltpu.sync_copy(r0, out.at[pl.ds(e0, _C)])

    def pair(jj, carry):
        j0 = 2 * jj
        e0 = pl.multiple_of((wid + j0 * _NW) * _C, 8)
        e1 = pl.multiple_of((wid + (j0 + 1) * _NW) * _C, 8)
        i0 = pltpu.async_copy(idx.at[pl.ds(e0, _C)], ic0, si0)
        i1 = pltpu.async_copy(idx.at[pl.ds(e1, _C)], ic1, si1)
        i0.wait()
        g0 = pltpu.async_copy(tab.at[ic0], r0, s0)
        i1.wait()
        g1 = pltpu.async_copy(tab.at[ic1], r1, s1)
        g0.wait()
        wa = pltpu.async_copy(r0, out.at[pl.ds(e0, _C)], w0)
        g1.wait()
        wb = pltpu.async_copy(r1, out.at[pl.ds(e1, _C)], w1)
        wa.wait()
        wb.wait()
        return carry

    lax.fori_loop(0, nk // 2, pair, 0)

    @pl.when(nk % 2 == 1)
    def _():
        chunk1(nk - 1)


_gather = pl.kernel(
    _gather_body,
    out_type=jax.ShapeDtypeStruct((_E, _D), jnp.float32),
    mesh=_mesh,
    scratch_types=[
        pltpu.VMEM((_C,), jnp.int32),
        pltpu.VMEM((_C,), jnp.int32),
        pltpu.VMEM((_C, _D), jnp.float32),
        pltpu.VMEM((_C, _D), jnp.float32),
        pltpu.SemaphoreType.DMA,
        pltpu.SemaphoreType.DMA,
        pltpu.SemaphoreType.DMA,
        pltpu.SemaphoreType.DMA,
        pltpu.SemaphoreType.DMA,
        pltpu.SemaphoreType.DMA,
    ],
)


def _seg_phase1(mrows, dstidx, c, s, ic0, ic1, r0, r1, s0, s1, a0, a1, si0, si1, macc):
    row0, nch = _own_rows(s)
    _fill(r0, _WB, _CH, 0.0)

    def zstep(k, carry):
        pltpu.sync_copy(r0.at[pl.ds(0, _WB)], macc.at[pl.ds(pl.multiple_of(row0 + k * _WB, 8), _WB)])
        return carry

    lax.fori_loop(0, nch, zstep, 0)
    plsc.subcore_barrier()

    ccol = c * _CH
    nk = _NCHK // _NS + jnp.where(s < _NCHK % _NS, 1, 0)

    def chunk1(j):
        e0 = pl.multiple_of((s + j * _NS) * _C, 8)
        pltpu.sync_copy(mrows.at[pl.ds(e0, _C), pl.ds(ccol, _CH)], r0)
        pltpu.sync_copy(dstidx.at[pl.ds(e0, _C)], ic0)
        pltpu.sync_copy(r0, macc.at[ic0], add=True)

    def pair(jj, carry):
        j0 = 2 * jj
        e0 = pl.multiple_of((s + j0 * _NS) * _C, 8)
        e1 = pl.multiple_of((s + (j0 + 1) * _NS) * _C, 8)
        i0 = pltpu.async_copy(dstidx.at[pl.ds(e0, _C)], ic0, si0)
        i1 = pltpu.async_copy(dstidx.at[pl.ds(e1, _C)], ic1, si1)
        l0 = pltpu.async_copy(mrows.at[pl.ds(e0, _C), pl.ds(ccol, _CH)], r0, s0)
        l1 = pltpu.async_copy(mrows.at[pl.ds(e1, _C), pl.ds(ccol, _CH)], r1, s1)
        i0.wait()
        l0.wait()
        x0 = pltpu.async_copy(r0, macc.at[ic0], a0, add=True)
        i1.wait()
        l1.wait()
        x1 = pltpu.async_copy(r1, macc.at[ic1], a1, add=True)
        x0.wait()
        x1.wait()
        return carry

    lax.fori_loop(0, nk // 2, pair, 0)

    @pl.when(nk % 2 == 1)
    def _():
        chunk1(nk - 1)

    plsc.subcore_barrier()


def _seg_writeback(out, c, s, r0, macc):
    row0, nch = _own_rows(s)

    def wstep(k, carry):
        rr = pl.multiple_of(row0 + k * _WB, 8)
        pltpu.sync_copy(macc.at[pl.ds(rr, _WB)], r0.at[pl.ds(0, _WB)])
        pltpu.sync_copy(r0.at[pl.ds(0, _WB)], out.at[pl.ds(rr, _WB), pl.ds(c * _CH, _CH)])
        return carry

    lax.fori_loop(0, nch, wstep, 0)


def _segsum_body(mrows, dstidx, out, ic0, ic1, r0, r1, s0, s1, a0, a1, si0, si1, macc):
    c = lax.axis_index("c")
    s = lax.axis_index("s")
    _seg_phase1(mrows, dstidx, c, s, ic0, ic1, r0, r1, s0, s1, a0, a1, si0, si1, macc)
    _seg_writeback(out, c, s, r0, macc)


_seg_scratch = [
    pltpu.VMEM((_C,), jnp.int32),
    pltpu.VMEM((_C,), jnp.int32),
    pltpu.VMEM((_C, _CH), jnp.float32),
    pltpu.VMEM((_C, _CH), jnp.float32),
    pltpu.SemaphoreType.DMA,
    pltpu.SemaphoreType.DMA,
    pltpu.SemaphoreType.DMA,
    pltpu.SemaphoreType.DMA,
    pltpu.SemaphoreType.DMA,
    pltpu.SemaphoreType.DMA,
    pltpu.VMEM_SHARED((_N, _CH), jnp.float32),
]

_segsum = pl.kernel(
    _segsum_body,
    out_type=jax.ShapeDtypeStruct((_N, _D), jnp.float32),
    mesh=_mesh,
    scratch_types=_seg_scratch,
)


def _segsum_gather_body(mrows, dstidx, srcidx, gmout,
                        ic0, ic1, r0, r1, s0, s1, a0, a1, si0, si1, macc):
    c = lax.axis_index("c")
    s = lax.axis_index("s")
    _seg_phase1(mrows, dstidx, c, s, ic0, ic1, r0, r1, s0, s1, a0, a1, si0, si1, macc)

    ccol = c * _CH
    nk = _NCHK // _NS + jnp.where(s < _NCHK % _NS, 1, 0)

    def chunk1(j):
        e0 = pl.multiple_of((s + j * _NS) * _C, 8)
        pltpu.sync_copy(srcidx.at[pl.ds(e0, _C)], ic0)
        pltpu.async_copy(macc.at[ic0], r0, s0).wait()
        pltpu.sync_copy(r0, gmout.at[pl.ds(e0, _C), pl.ds(ccol, _CH)])

    def pair(jj, carry):
        j0 = 2 * jj
        e0 = pl.multiple_of((s + j0 * _NS) * _C, 8)
        e1 = pl.multiple_of((s + (j0 + 1) * _NS) * _C, 8)
        i0 = pltpu.async_copy(srcidx.at[pl.ds(e0, _C)], ic0, si0)
        i1 = pltpu.async_copy(srcidx.at[pl.ds(e1, _C)], ic1, si1)
        i0.wait()
        g0 = pltpu.async_copy(macc.at[ic0], r0, s0)
        i1.wait()
        g1 = pltpu.async_copy(macc.at[ic1], r1, s1)
        g0.wait()
        wa = pltpu.async_copy(r0, gmout.at[pl.ds(e0, _C), pl.ds(ccol, _CH)], a0)
        g1.wait()
        wb = pltpu.async_copy(r1, gmout.at[pl.ds(e1, _C), pl.ds(ccol, _CH)], a1)
        wa.wait()
        wb.wait()
        return carry

    lax.fori_loop(0, nk // 2, pair, 0)

    @pl.when(nk % 2 == 1)
    def _():
        chunk1(nk - 1)


_segsum_gather = pl.kernel(
    _segsum_gather_body,
    out_type=jax.ShapeDtypeStruct((_E, _D), jnp.float32),
    mesh=_mesh,
    scratch_types=_seg_scratch,
)


def _deg_body(srcidx, out, gdout, ic0, ic1, icg0, icg1, ones, gr0, gr1, s0, s1, a0, a1, si0, si1, dacc):
    c = lax.axis_index("c")
    s = lax.axis_index("s")
    row0, nch = _own_rows(s)
    _fill(gr0, _WB, _DG, 0.0)
    _fill(ones, _C, _DG, 1.0)

    def zstep(k, carry):
        pltpu.sync_copy(gr0.at[pl.ds(0, _WB)], dacc.at[pl.ds(pl.multiple_of(row0 + k * _WB, 8), _WB)])
        return carry

    lax.fori_loop(0, nch, zstep, 0)
    plsc.subcore_barrier()

    nk = _NCHK // _NS + jnp.where(s < _NCHK % _NS, 1, 0)

    def schunk1(j):
        e0 = pl.multiple_of((s + j * _NS) * _C, 8)
        pltpu.sync_copy(srcidx.at[pl.ds(e0, _C)], ic0)
        pltpu.sync_copy(ones, dacc.at[ic0], add=True)

    def pair(jj, carry):
        j0 = 2 * jj
        e0 = pl.multiple_of((s + j0 * _NS) * _C, 8)
        e1 = pl.multiple_of((s + (j0 + 1) * _NS) * _C, 8)
        i0 = pltpu.async_copy(srcidx.at[pl.ds(e0, _C)], ic0, si0)
        i1 = pltpu.async_copy(srcidx.at[pl.ds(e1, _C)], ic1, si1)
        i0.wait()
        x0 = pltpu.async_copy(ones, dacc.at[ic0], a0, add=True)
        i1.wait()
        x1 = pltpu.async_copy(ones, dacc.at[ic1], a1, add=True)
        x0.wait()
        x1.wait()
        return carry

    lax.fori_loop(0, nk // 2, pair, 0)

    @pl.when(nk % 2 == 1)
    def _():
        schunk1(nk - 1)

    plsc.subcore_barrier()

    wid = s * _NC + c
    gnk = _NCHK // _NW + jnp.where(wid < _NCHK % _NW, 1, 0)

    def gchunk1(j):
        e0 = pl.multiple_of((wid + j * _NW) * _C, 8)
        pltpu.sync_copy(srcidx.at[pl.ds(e0, _C)], icg0)
        pltpu.async_copy(dacc.at[icg0], gr0, s0).wait()
        pltpu.sync_copy(gr0, gdout.at[pl.ds(e0, _C)])

    def gpair(jj, carry):
        j0 = 2 * jj
        e0 = pl.multiple_of((wid + j0 * _NW) * _C, 8)
        e1 = pl.multiple_of((wid + (j0 + 1) * _NW) * _C, 8)
        i0 = pltpu.async_copy(srcidx.at[pl.ds(e0, _C)], icg0, si0)
        i1 = pltpu.async_copy(srcidx.at[pl.ds(e1, _C)], icg1, si1)
        i0.wait()
        g0 = pltpu.async_copy(dacc.at[icg0], gr0, s0)
        i1.wait()
        g1 = pltpu.async_copy(dacc.at[icg1], gr1, s1)
        g0.wait()
        wa = pltpu.async_copy(gr0, gdout.at[pl.ds(e0, _C)], a0)
        g1.wait()
        wb = pltpu.async_copy(gr1, gdout.at[pl.ds(e1, _C)], a1)
        wa.wait()
        wb.wait()
        return carry

    lax.fori_loop(0, gnk // 2, gpair, 0)

    @pl.when(gnk % 2 == 1)
    def _():
        gchunk1(gnk - 1)

    @pl.when(c == 0)
    def _():
        def wstep(k, carry):
            rr = pl.multiple_of(row0 + k * _WB, 8)
            pltpu.sync_copy(dacc.at[pl.ds(rr, _WB)], gr0.at[pl.ds(0, _WB)])
            pltpu.sync_copy(gr0.at[pl.ds(0, _WB)], out.at[pl.ds(rr, _WB)])
            return carry

        lax.fori_loop(0, nch, wstep, 0)


_deg = pl.kernel(
    _deg_body,
    out_type=(
        jax.ShapeDtypeStruct((_N, _DG), jnp.float32),
        jax.ShapeDtypeStruct((_E, _DG), jnp.float32),
    ),
    mesh=_mesh,
    scratch_types=[
        pltpu.VMEM((_C,), jnp.int32),
        pltpu.VMEM((_C,), jnp.int32),
        pltpu.VMEM((_C,), jnp.int32),
        pltpu.VMEM((_C,), jnp.int32),
        pltpu.VMEM((_C, _DG), jnp.float32),
        pltpu.VMEM((_C, _DG), jnp.float32),
        pltpu.VMEM((_C, _DG), jnp.float32),
        pltpu.SemaphoreType.DMA,
        pltpu.SemaphoreType.DMA,
        pltpu.SemaphoreType.DMA,
        pltpu.SemaphoreType.DMA,
        pltpu.SemaphoreType.DMA,
        pltpu.SemaphoreType.DMA,
        pltpu.VMEM_SHARED((_N, _DG), jnp.float32),
    ],
)


_BLK_N = 2000
_CN = (((1,), (1,)), ((), ()))


def _mlp_body(xb, w1, w2, wp, bpb, ob):
    xx = xb[...]
    a = lax.dot_general(xx, w1[...], _CN, preferred_element_type=jnp.float32)
    g = lax.dot_general(xx, w2[...], _CN, preferred_element_type=jnp.float32)
    hh = (a * jax.nn.sigmoid(a)) * g
    ob[...] = lax.dot_general(hh, wp[...], _CN, preferred_element_type=jnp.float32) + bpb[...]


def _mlp(x, W1, W2, Wp, bp):
    return pl.pallas_call(
        _mlp_body,
        grid=(_N // _BLK_N,),
        in_specs=[
            pl.BlockSpec((_BLK_N, _D), lambda i: (i, 0)),
            pl.BlockSpec((_HID, _D), lambda i: (0, 0)),
            pl.BlockSpec((_HID, _D), lambda i: (0, 0)),
            pl.BlockSpec((_D, _HID), lambda i: (0, 0)),
            pl.BlockSpec((1, _D), lambda i: (0, 0)),
        ],
        out_specs=pl.BlockSpec((_BLK_N, _D), lambda i: (i, 0)),
        out_shape=jax.ShapeDtypeStruct((_N, _D), jnp.float32),
    )(x, W1, W2, Wp, bp.reshape(1, _D))


def _swap(v):
    ev = (lax.broadcasted_iota(jnp.int32, v.shape, 1) % 2) == 0
    return jnp.where(ev, pltpu.roll(v, _D - 1, 1), pltpu.roll(v, 1, 1))


_BLK_E = 3200
_NB_E = _E // _BLK_E
_HALF = (_E // 2) // _BLK_E


def _combine_body(gh, gm, mprev, gdeg, ca, sa, cb, sb, out):
    xe = gh[...]
    de = gdeg[...][:, :1]
    dm = gm[...] - mprev[...]
    la = ca[...] * xe + sa[...] * _swap(xe)
    lb = cb[...] * dm + sb[...] * _swap(dm)
    out[...] = jnp.where(de == 1.0, xe, la + lb / (de - 1.0 + 1e-9))


def _combine(gh, gm, mprev, gdeg, CA, SA, CB, SB):
    coef = pl.BlockSpec((1, _D), lambda i: (0, 0))
    eb = pl.BlockSpec((_BLK_E, _D), lambda i: (i, 0))
    return pl.pallas_call(
        _combine_body,
        grid=(_NB_E,),
        in_specs=[
            eb, eb,
            pl.BlockSpec((_BLK_E, _D), lambda i: ((i + _HALF) % _NB_E, 0)),
            pl.BlockSpec((_BLK_E, _DG), lambda i: (i, 0)),
            coef, coef, coef, coef,
        ],
        out_specs=eb,
        out_shape=jax.ShapeDtypeStruct((_E, _D), jnp.float32),
    )(gh, gm, mprev, gdeg, CA, SA, CB, SB)


def _final_body(xb, hb, mb, degb, ca, sa, cb, sb, out):
    hh = hb[...]
    mm = mb[...]
    dg = degb[...][:, :1]
    la = ca[...] * hh + sa[...] * _swap(hh)
    lb = cb[...] * mm + sb[...] * _swap(mm)
    val = jnp.where(dg == 0.0, hh, la + lb / (dg + 1e-9))
    out[...] = xb[...] + jnp.maximum(val, 0.0)


def _final(x, h, m, degn, CA, SA, CB, SB):
    coef = pl.BlockSpec((1, _D), lambda i: (0, 0))
    nb = pl.BlockSpec((_BLK_N, _D), lambda i: (i, 0))
    return pl.pallas_call(
        _final_body,
        grid=(_N // _BLK_N,),
        in_specs=[
            nb, nb, nb,
            pl.BlockSpec((_BLK_N, _DG), lambda i: (i, 0)),
            coef, coef, coef, coef,
        ],
        out_specs=nb,
        out_shape=jax.ShapeDtypeStruct((_N, _D), jnp.float32),
    )(x, h, m, degn, CA, SA, CB, SB)


def kernel(x, edge_index, log_dt, log_lambda_real, lambda_imag, W1, W2, Wp, bp):
    src = edge_index[0]
    dst = edge_index[1]

    dt = jnp.exp(log_dt)
    mag = jnp.exp(-jnp.exp(log_lambda_real) * dt)
    ph = lambda_imag * dt
    cw = jnp.cos(ph)
    sw = jnp.sin(ph)

    def coeffs(mx):
        cc = jnp.repeat(mx * cw, 2).reshape(1, _D)
        ss = jnp.stack((-mx * sw, mx * sw), axis=-1).reshape(1, _D)
        return cc, ss

    CA, SA = coeffs(1.0 - mag)
    CB, SB = coeffs(mag)

    h = _mlp(x, W1, W2, Wp, bp)
    degn, gdeg = _deg(src)
    gh = _gather(h, src)

    M = gh
    gm = _segsum_gather(gh, dst, src)
    for t in range(_T):
        M = _combine(gh, gm, M, gdeg, CA, SA, CB, SB)
        if t < _T - 1:
            gm = _segsum_gather(M, dst, src)
        else:
            m = _segsum(M, dst)

    return _final(x, h, m, degn, CA, SA, CB, SB)
